# baseline (speedup 1.0000x reference)
import functools
import math

import jax
import jax.numpy as jnp
from jax import lax
from jax.experimental import pallas as pl
from jax.experimental.pallas import tpu as pltpu
from jax.experimental.pallas import tpu_sc as plsc

S, D = 2048, 672
H, KVH, HD = 12, 4, 56
E, DFF, TOPK = 8, 2048, 2
EPS = 1e-05
HALF = HD // 2
GRP = H // KVH

SBLK = 256
NSB = S // SBLK
KBLK = 256
BLK_M = 256
NBUF = TOPK * S + E * BLK_M
NB = NBUF // BLK_M
DCH = 512
NDC = DFF // DCH
NW = 32
DP = 768


def _rmsn(x, scale):
    return x * lax.rsqrt(jnp.mean(x * x, axis=-1, keepdims=True) + EPS) * scale


def _swap_halves(t):
    n, w = t.shape
    z = jnp.zeros((n, HALF), t.dtype)
    lo = jnp.concatenate([t[:, HALF:], z], axis=1)
    hi = jnp.concatenate([z, t[:, :w - HALF]], axis=1)
    lane = lax.broadcasted_iota(jnp.int32, t.shape, 1)
    return jnp.where(lane % HD < HALF, lo, hi)


def _head_rms(q, gmat, gmat_t, scale_f):
    gs = jnp.dot(q * q, gmat, preferred_element_type=jnp.float32)
    scl = lax.rsqrt(gs * (1.0 / HD) + EPS)
    return q * jnp.dot(scl, gmat_t, preferred_element_type=jnp.float32) * scale_f


def _q_body(x_ref, s1_ref, wq_ref, qs_ref, g_ref, gt_ref, cf_ref, sf_ref,
            qo_ref):
    h = _rmsn(x_ref[...], s1_ref[...]).astype(jnp.bfloat16)
    q = jnp.dot(h, wq_ref[...], preferred_element_type=jnp.float32)
    q = _head_rms(q, g_ref[...], gt_ref[...], qs_ref[...])
    qr = (q * cf_ref[...] + _swap_halves(q) * sf_ref[...]).astype(jnp.bfloat16)
    for hh in range(H):
        qo_ref[hh] = qr[:, HD * hh:HD * (hh + 1)]


def _q_proj(x2, s1, wq, qsf, gq, gqt, cfq, sfq):
    return pl.pallas_call(
        _q_body,
        grid=(NSB,),
        in_specs=[
            pl.BlockSpec((SBLK, D), lambda i: (i, 0)),
            pl.BlockSpec((1, D), lambda i: (0, 0)),
            pl.BlockSpec((D, H * HD), lambda i: (0, 0)),
            pl.BlockSpec((1, H * HD), lambda i: (0, 0)),
            pl.BlockSpec((H * HD, H), lambda i: (0, 0)),
            pl.BlockSpec((H, H * HD), lambda i: (0, 0)),
            pl.BlockSpec((SBLK, H * HD), lambda i: (i, 0)),
            pl.BlockSpec((SBLK, H * HD), lambda i: (i, 0)),
        ],
        out_specs=pl.BlockSpec((H, SBLK, HD), lambda i: (0, i, 0)),
        out_shape=jax.ShapeDtypeStruct((H, S, HD), jnp.bfloat16),
    )(x2, s1, wq.astype(jnp.bfloat16), qsf, gq, gqt, cfq, sfq)


def _kv_body(x_ref, s1_ref, wk_ref, wv_ref, ks_ref, g_ref, gt_ref,
             cf_ref, sf_ref, ko_ref, vo_ref):
    h = _rmsn(x_ref[...], s1_ref[...]).astype(jnp.bfloat16)
    k = jnp.dot(h, wk_ref[...], preferred_element_type=jnp.float32)
    k = _head_rms(k, g_ref[...], gt_ref[...], ks_ref[...])
    kr = (k * cf_ref[...] + _swap_halves(k) * sf_ref[...]).astype(jnp.bfloat16)
    v = jnp.dot(h, wv_ref[...], preferred_element_type=jnp.float32)
    vb = v.astype(jnp.bfloat16)
    for hh in range(KVH):
        ko_ref[hh] = kr[:, HD * hh:HD * (hh + 1)]
        vo_ref[hh] = vb[:, HD * hh:HD * (hh + 1)]


def _kv_proj(x2, s1, wk, wv, ksf, gk, gkt, cfk, sfk):
    kvd = KVH * HD
    return pl.pallas_call(
        _kv_body,
        grid=(NSB,),
        in_specs=[
            pl.BlockSpec((SBLK, D), lambda i: (i, 0)),
            pl.BlockSpec((1, D), lambda i: (0, 0)),
            pl.BlockSpec((D, kvd), lambda i: (0, 0)),
            pl.BlockSpec((D, kvd), lambda i: (0, 0)),
            pl.BlockSpec((1, kvd), lambda i: (0, 0)),
            pl.BlockSpec((kvd, KVH), lambda i: (0, 0)),
            pl.BlockSpec((KVH, kvd), lambda i: (0, 0)),
            pl.BlockSpec((SBLK, kvd), lambda i: (i, 0)),
            pl.BlockSpec((SBLK, kvd), lambda i: (i, 0)),
        ],
        out_specs=[
            pl.BlockSpec((KVH, SBLK, HD), lambda i: (0, i, 0)),
            pl.BlockSpec((KVH, SBLK, HD), lambda i: (0, i, 0)),
        ],
        out_shape=[
            jax.ShapeDtypeStruct((KVH, S, HD), jnp.bfloat16),
            jax.ShapeDtypeStruct((KVH, S, HD), jnp.bfloat16),
        ],
    )(x2, s1, wk.astype(jnp.bfloat16), wv.astype(jnp.bfloat16), ksf,
      gk, gkt, cfk, sfk)


def _attn_body(q_ref, k_ref, v_ref, o_ref):
    i = pl.program_id(1)
    q = q_ref[0]
    scale = 1.0 / math.sqrt(HD)

    def step(kb, carry):
        m, l, acc = carry
        kb0 = kb * KBLK
        kblk = k_ref[0, pl.ds(kb0, KBLK), :]
        vblk = v_ref[0, pl.ds(kb0, KBLK), :]
        s = lax.dot_general(q, kblk, (((1,), (1,)), ((), ())),
                            preferred_element_type=jnp.float32) * scale
        qpos = i * SBLK + lax.broadcasted_iota(jnp.int32, (SBLK, KBLK), 0)
        kpos = kb0 + lax.broadcasted_iota(jnp.int32, (SBLK, KBLK), 1)
        s = jnp.where(qpos >= kpos, s, -1e30)
        mn = jnp.maximum(m, jnp.max(s, axis=-1, keepdims=True))
        p = jnp.exp(s - mn)
        corr = jnp.exp(m - mn)
        l = l * corr + jnp.sum(p, axis=-1, keepdims=True)
        acc = acc * corr + jnp.dot(p.astype(jnp.bfloat16), vblk,
                                   preferred_element_type=jnp.float32)
        return mn, l, acc

    m0 = jnp.full((SBLK, 1), -1e30, jnp.float32)
    l0 = jnp.zeros((SBLK, 1), jnp.float32)
    a0 = jnp.zeros((SBLK, HD), jnp.float32)
    m, l, acc = lax.fori_loop(0, i + 1, step, (m0, l0, a0))
    o_ref[0] = (acc / l).astype(jnp.bfloat16)


def _attention(q3, k3, v3):
    return pl.pallas_call(
        _attn_body,
        grid=(H, NSB),
        in_specs=[
            pl.BlockSpec((1, SBLK, HD), lambda h, i: (h, i, 0)),
            pl.BlockSpec((1, S, HD), lambda h, i: (h // GRP, 0, 0)),
            pl.BlockSpec((1, S, HD), lambda h, i: (h // GRP, 0, 0)),
        ],
        out_specs=pl.BlockSpec((1, SBLK, HD), lambda h, i: (h, i, 0)),
        out_shape=jax.ShapeDtypeStruct((H, S, HD), jnp.bfloat16),
    )(q3, k3, v3)


def _post_body(o_ref, wo_ref, x_ref, s2_ref, rw_ref, rb_ref,
               x1_ref, h2_ref, rt_ref):
    o_flat = jnp.concatenate([o_ref[hh] for hh in range(H)], axis=-1)
    x1 = x_ref[...] + jnp.dot(o_flat, wo_ref[...],
                              preferred_element_type=jnp.float32)
    x1_ref[...] = x1
    h2 = _rmsn(x1, s2_ref[...])
    h2_ref[...] = jnp.concatenate(
        [h2, jnp.zeros((SBLK, DP - D), jnp.float32)], axis=-1)
    logits = (jnp.dot(h2, rw_ref[...], preferred_element_type=jnp.float32)
              + rb_ref[...])
    eio = lax.broadcasted_iota(jnp.int32, (SBLK, E), 1)
    m1 = jnp.max(logits, axis=-1, keepdims=True)
    i1 = jnp.min(jnp.where(logits == m1, eio, E), axis=-1, keepdims=True)
    l2 = jnp.where(eio == i1, -jnp.inf, logits)
    m2 = jnp.max(l2, axis=-1, keepdims=True)
    i2 = jnp.min(jnp.where(l2 == m2, eio, E), axis=-1, keepdims=True)
    g0 = 1.0 / (1.0 + jnp.exp(m2 - m1))
    g1 = 1.0 - g0
    z = jnp.zeros((SBLK, E - 4), jnp.float32)
    rt_ref[...] = jnp.concatenate(
        [i1.astype(jnp.float32), i2.astype(jnp.float32), g0, g1, z],
        axis=-1)


def _post_attn(o3, wo, x2, s2, rw, rb):
    return pl.pallas_call(
        _post_body,
        grid=(NSB,),
        in_specs=[
            pl.BlockSpec((H, SBLK, HD), lambda i: (0, i, 0)),
            pl.BlockSpec((H * HD, D), lambda i: (0, 0)),
            pl.BlockSpec((SBLK, D), lambda i: (i, 0)),
            pl.BlockSpec((1, D), lambda i: (0, 0)),
            pl.BlockSpec((D, E), lambda i: (0, 0)),
            pl.BlockSpec((1, E), lambda i: (0, 0)),
        ],
        out_specs=[
            pl.BlockSpec((SBLK, D), lambda i: (i, 0)),
            pl.BlockSpec((SBLK, DP), lambda i: (i, 0)),
            pl.BlockSpec((SBLK, E), lambda i: (i, 0)),
        ],
        out_shape=[
            jax.ShapeDtypeStruct((S, D), jnp.float32),
            jax.ShapeDtypeStruct((S, DP), jnp.float32),
            jax.ShapeDtypeStruct((S, E), jnp.float32),
        ],
    )(o3, wo.astype(jnp.bfloat16), x2, s2, rw, rb)


def _excl_cumsum_rows(x):
    acc = x
    sh = 1
    while sh < S:
        acc = acc + jnp.concatenate(
            [jnp.zeros((sh, E), x.dtype), acc[:S - sh, :]], axis=0)
        sh *= 2
    return acc - x


def _excl_cumsum_lanes(x):
    acc = x
    for sh in (1, 2, 4):
        acc = acc + jnp.concatenate(
            [jnp.zeros((1, sh), x.dtype), acc[:, :E - sh]], axis=1)
    return acc - x


def _route_body(rt_ref, pos_ref, binfo_ref):
    rt = rt_ref[...]
    e0 = rt[:, 0:1].astype(jnp.int32)
    e1 = rt[:, 1:2].astype(jnp.int32)
    eio = lax.broadcasted_iota(jnp.int32, (S, E), 1)
    oh0 = (e0 == eio).astype(jnp.float32)
    oh1 = (e1 == eio).astype(jnp.float32)
    c0 = _excl_cumsum_rows(oh0)
    c1 = _excl_cumsum_rows(oh1)
    tot0 = jnp.sum(oh0, axis=0, keepdims=True)
    sizes = tot0 + jnp.sum(oh1, axis=0, keepdims=True)
    padded = jnp.floor((sizes + (BLK_M - 1)) / BLK_M) * BLK_M
    poff = _excl_cumsum_lanes(padded)
    pos0 = jnp.sum(oh0 * (poff + c0), axis=-1, keepdims=True)
    pos1 = jnp.sum(oh1 * (poff + tot0 + c1), axis=-1, keepdims=True)
    zs = jnp.zeros((S, E - 2), jnp.float32)
    pos_ref[...] = jnp.concatenate([pos0, pos1, zs], axis=-1).astype(jnp.int32)

    bstart = poff / BLK_M
    pb = padded / BLK_M
    nused = jnp.sum(pb)
    eiof = lax.broadcasted_iota(jnp.int32, (1, E), 1).astype(jnp.float32)
    belast = jnp.max(jnp.where(pb > 0, eiof, 0.0))
    bio = lax.broadcasted_iota(jnp.int32, (1, 128), 1).astype(jnp.float32)
    bi = jnp.zeros((1, 128), jnp.float32)
    for e in range(E):
        m = (bio >= bstart[0, e]) & (bio < bstart[0, e] + pb[0, e])
        bi = jnp.where(m, float(e), bi)
    bi = jnp.where((bio >= nused) & (bio < float(NB)), belast, bi)
    bi = jnp.where(bio == 32.0, nused, bi)
    binfo_ref[...] = bi.astype(jnp.int32)


def _route_plan(rt):
    return pl.pallas_call(
        _route_body,
        out_shape=[
            jax.ShapeDtypeStruct((S, E), jnp.int32),
            jax.ShapeDtypeStruct((1, 128), jnp.int32),
        ],
    )(rt)


_APW = TOPK * S // NW


@functools.lru_cache(maxsize=None)
def _sc_mesh():
    return plsc.VectorSubcoreMesh(core_axis_name="c", subcore_axis_name="s")


@functools.lru_cache(maxsize=None)
def _build_sc_dispatch():
    @functools.partial(
        pl.kernel,
        mesh=_sc_mesh(),
        out_type=jax.ShapeDtypeStruct((NBUF, DP), jnp.float32),
        scratch_types=[
            pltpu.VMEM((_APW,), jnp.int32),
            pltpu.VMEM((_APW, DP), jnp.float32),
            pltpu.SemaphoreType.DMA,
        ],
    )
    def dispatch(pos_hbm, h2_hbm, xs_hbm, idx_v, rows_v, sem):
        wid = lax.axis_index("s") * 2 + lax.axis_index("c")
        base = wid * _APW
        tbase = lax.rem(base, S)
        pltpu.sync_copy(pos_hbm.at[pl.ds(base, _APW)], idx_v)
        pltpu.sync_copy(h2_hbm.at[pl.ds(tbase, _APW)], rows_v)
        pltpu.async_copy(rows_v, xs_hbm.at[idx_v], sem).wait()

    return dispatch


def _sc_dispatch(pos_flat, h2):
    return _build_sc_dispatch()(pos_flat, h2)


_TPW = S // NW
_CH = 32


@functools.lru_cache(maxsize=None)
def _build_sc_combine():
    @functools.partial(
        pl.kernel,
        mesh=_sc_mesh(),
        out_type=(
            jax.ShapeDtypeStruct((S, DP), jnp.float32),
            jax.ShapeDtypeStruct((S, DP), jnp.float32),
        ),
        scratch_types=[
            pltpu.VMEM((_CH,), jnp.int32),
            pltpu.VMEM((_CH,), jnp.int32),
            pltpu.VMEM((_CH, DP), jnp.float32),
            pltpu.VMEM((_CH, DP), jnp.float32),
            pltpu.SemaphoreType.DMA,
            pltpu.SemaphoreType.DMA,
        ],
    )
    def combine(p0_hbm, p1_hbm, eo_hbm, r0_hbm, r1_hbm,
                i0_v, i1_v, r0_v, r1_v, s0, s1):
        wid = lax.axis_index("s") * 2 + lax.axis_index("c")
        for c in range(_TPW // _CH):
            base = wid * _TPW + c * _CH
            pltpu.sync_copy(p0_hbm.at[pl.ds(base, _CH)], i0_v)
            pltpu.sync_copy(p1_hbm.at[pl.ds(base, _CH)], i1_v)
            a = pltpu.async_copy(eo_hbm.at[i0_v], r0_v, s0)
            b = pltpu.async_copy(eo_hbm.at[i1_v], r1_v, s1)
            a.wait()
            b.wait()
            pltpu.sync_copy(r0_v, r0_hbm.at[pl.ds(base, _CH)])
            pltpu.sync_copy(r1_v, r1_hbm.at[pl.ds(base, _CH)])

    return combine


def _sc_combine(pos0, pos1, eo):
    return _build_sc_combine()(pos0, pos1, eo)


def _ffn_body(b_ref, xs_ref, w1_ref, b1_ref, w2_ref, b2_ref, eo_ref):
    i = pl.program_id(0)
    nused = b_ref[32]

    @pl.when(i < nused)
    def _():
        hid = (jnp.dot(xs_ref[...][:, :D].astype(jnp.bfloat16), w1_ref[0],
                       preferred_element_type=jnp.float32) + b1_ref[0])
        act = jnp.square(jnp.maximum(hid, 0.0)).astype(jnp.bfloat16)
        contrib = (jnp.dot(act, w2_ref[0], preferred_element_type=jnp.float32)
                   + b2_ref[0])
        zpad = jnp.zeros((BLK_M, DP - D), jnp.float32)
        eo_ref[...] = jnp.concatenate([contrib, zpad], axis=-1)


def _expert_ffn(binfo, xs, w1, b1, w2, b2):
    grid_spec = pltpu.PrefetchScalarGridSpec(
        num_scalar_prefetch=1,
        grid=(NB,),
        in_specs=[
            pl.BlockSpec((BLK_M, DP), lambda i, b: (i, 0)),
            pl.BlockSpec((1, D, DFF), lambda i, b: (b[i], 0, 0)),
            pl.BlockSpec((1, 1, DFF), lambda i, b: (b[i], 0, 0)),
            pl.BlockSpec((1, DFF, D), lambda i, b: (b[i], 0, 0)),
            pl.BlockSpec((1, 1, D), lambda i, b: (b[i], 0, 0)),
        ],
        out_specs=pl.BlockSpec((BLK_M, DP), lambda i, b: (i, 0)),
    )
    return pl.pallas_call(
        _ffn_body,
        grid_spec=grid_spec,
        out_shape=jax.ShapeDtypeStruct((NBUF, DP), jnp.float32),
    )(binfo, xs, w1.astype(jnp.bfloat16), b1.reshape(E, 1, DFF),
      w2.astype(jnp.bfloat16), b2.reshape(E, 1, D))


def _comb_body(x1_ref, r0_ref, r1_ref, rt_ref, y_ref):
    g0 = rt_ref[:, 2:3]
    g1 = rt_ref[:, 3:4]
    y_ref[...] = (x1_ref[...] + g0 * r0_ref[...][:, :D]
                  + g1 * r1_ref[...][:, :D])


def _final_combine(x1, r0, r1, rt):
    return pl.pallas_call(
        _comb_body,
        grid=(NSB,),
        in_specs=[
            pl.BlockSpec((SBLK, D), lambda i: (i, 0)),
            pl.BlockSpec((SBLK, DP), lambda i: (i, 0)),
            pl.BlockSpec((SBLK, DP), lambda i: (i, 0)),
            pl.BlockSpec((SBLK, E), lambda i: (i, 0)),
        ],
        out_specs=pl.BlockSpec((SBLK, D), lambda i: (i, 0)),
        out_shape=jax.ShapeDtypeStruct((S, D), jnp.float32),
    )(x1, r0, r1, rt)


def kernel(x, rms1_scale, wq, wk, wv, wo, q_norm_scale, k_norm_scale,
           rms2_scale, router_w, router_b, w1, b1, w2, b2):
    x2 = x.reshape(S, D)
    s1 = rms1_scale.reshape(1, D)
    s2 = rms2_scale.reshape(1, D)
    rb = router_b.reshape(1, E)
    qsf = jnp.tile(q_norm_scale, H).reshape(1, H * HD)
    ksf = jnp.tile(k_norm_scale, KVH).reshape(1, KVH * HD)
    gq = jnp.repeat(jnp.eye(H, dtype=jnp.float32), HD, axis=0)
    gk = jnp.repeat(jnp.eye(KVH, dtype=jnp.float32), HD, axis=0)

    inv = 1.0 / (10000.0 ** (jnp.arange(HALF, dtype=jnp.float32) / HALF))
    ang = jnp.arange(S, dtype=jnp.float32)[:, None] * inv[None, :]
    cos = jnp.cos(ang)
    sin = jnp.sin(ang)
    cos2 = jnp.concatenate([cos, cos], axis=1)
    sin2 = jnp.concatenate([-sin, sin], axis=1)
    cfq = jnp.tile(cos2, (1, H))
    sfq = jnp.tile(sin2, (1, H))
    cfk = jnp.tile(cos2, (1, KVH))
    sfk = jnp.tile(sin2, (1, KVH))

    q3 = _q_proj(x2, s1, wq, qsf, gq, gq.T, cfq, sfq)
    k3, v3 = _kv_proj(x2, s1, wk, wv, ksf, gk, gk.T, cfk, sfk)
    o3 = _attention(q3, k3, v3)
    x1, h2, rt = _post_attn(o3, wo, x2, s2, router_w, rb)

    pos, binfo = _route_plan(rt)
    pos_flat = jnp.concatenate([pos[:, 0], pos[:, 1]], axis=0)
    xs = _sc_dispatch(pos_flat, h2)
    eo = _expert_ffn(binfo.reshape(128), xs, w1, b1, w2, b2)
    r0, r1 = _sc_combine(pos[:, 0], pos[:, 1], eo)
    y = _final_combine(x1, r0, r1, rt)
    return y.reshape(1, S, D)


# speedup vs baseline: 1.4666x; 1.4666x over previous
import functools
import math

import jax
import jax.numpy as jnp
from jax import lax
from jax.experimental import pallas as pl
from jax.experimental.pallas import tpu as pltpu
from jax.experimental.pallas import tpu_sc as plsc

S, D = 2048, 672
H, KVH, HD = 12, 4, 56
E, DFF, TOPK = 8, 2048, 2
EPS = 1e-05
HALF = HD // 2
GRP = H // KVH

SBLK = 256
NSB = S // SBLK
KBLK = 256
BLK_M = 256
NBUF = TOPK * S + E * BLK_M
NB = NBUF // BLK_M
DCH = 512
NDC = DFF // DCH
NW = 32
DP = 768


def _rmsn(x, scale):
    return x * lax.rsqrt(jnp.mean(x * x, axis=-1, keepdims=True) + EPS) * scale


def _swap_halves(t):
    n, w = t.shape
    z = jnp.zeros((n, HALF), t.dtype)
    lo = jnp.concatenate([t[:, HALF:], z], axis=1)
    hi = jnp.concatenate([z, t[:, :w - HALF]], axis=1)
    lane = lax.broadcasted_iota(jnp.int32, t.shape, 1)
    return jnp.where(lane % HD < HALF, lo, hi)


def _head_rms(q, gmat, gmat_t, scale_f):
    gs = jnp.dot(q * q, gmat, preferred_element_type=jnp.float32)
    scl = lax.rsqrt(gs * (1.0 / HD) + EPS)
    return q * jnp.dot(scl, gmat_t, preferred_element_type=jnp.float32) * scale_f


def _qkv_body(x_ref, s1_ref, wq_ref, wk_ref, wv_ref, qs_ref, ks_ref,
              gq_ref, gqt_ref, gk_ref, gkt_ref, cfq_ref, sfq_ref,
              cfk_ref, sfk_ref, qo_ref, ko_ref, vo_ref):
    h = _rmsn(x_ref[...], s1_ref[...])
    q = jnp.dot(h, wq_ref[...], preferred_element_type=jnp.float32)
    q = _head_rms(q, gq_ref[...], gqt_ref[...], qs_ref[...])
    qr = q * cfq_ref[...] + _swap_halves(q) * sfq_ref[...]
    for hh in range(H):
        qo_ref[hh] = qr[:, HD * hh:HD * (hh + 1)]
    k = jnp.dot(h, wk_ref[...], preferred_element_type=jnp.float32)
    k = _head_rms(k, gk_ref[...], gkt_ref[...], ks_ref[...])
    kr = k * cfk_ref[...] + _swap_halves(k) * sfk_ref[...]
    v = jnp.dot(h, wv_ref[...], preferred_element_type=jnp.float32)
    for hh in range(KVH):
        ko_ref[hh] = kr[:, HD * hh:HD * (hh + 1)]
        vo_ref[hh] = v[:, HD * hh:HD * (hh + 1)]


def _qkv_proj(x2, s1, wq, wk, wv, qsf, ksf, gq, gk, cfq, sfq, cfk, sfk):
    kvd = KVH * HD
    c0 = lambda i: (0, 0)
    return pl.pallas_call(
        _qkv_body,
        grid=(NSB,),
        in_specs=[
            pl.BlockSpec((SBLK, D), lambda i: (i, 0)),
            pl.BlockSpec((1, D), c0),
            pl.BlockSpec((D, H * HD), c0),
            pl.BlockSpec((D, kvd), c0),
            pl.BlockSpec((D, kvd), c0),
            pl.BlockSpec((1, H * HD), c0),
            pl.BlockSpec((1, kvd), c0),
            pl.BlockSpec((H * HD, H), c0),
            pl.BlockSpec((H, H * HD), c0),
            pl.BlockSpec((kvd, KVH), c0),
            pl.BlockSpec((KVH, kvd), c0),
            pl.BlockSpec((SBLK, H * HD), lambda i: (i, 0)),
            pl.BlockSpec((SBLK, H * HD), lambda i: (i, 0)),
            pl.BlockSpec((SBLK, kvd), lambda i: (i, 0)),
            pl.BlockSpec((SBLK, kvd), lambda i: (i, 0)),
        ],
        out_specs=[
            pl.BlockSpec((H, SBLK, HD), lambda i: (0, i, 0)),
            pl.BlockSpec((KVH, SBLK, HD), lambda i: (0, i, 0)),
            pl.BlockSpec((KVH, SBLK, HD), lambda i: (0, i, 0)),
        ],
        out_shape=[
            jax.ShapeDtypeStruct((H, S, HD), jnp.float32),
            jax.ShapeDtypeStruct((KVH, S, HD), jnp.float32),
            jax.ShapeDtypeStruct((KVH, S, HD), jnp.float32),
        ],
    )(x2, s1, wq, wk, wv, qsf, ksf, gq, gq.T, gk, gk.T, cfq, sfq, cfk, sfk)


QBLK = 512
KBLK = 512
NQB = S // QBLK


def _attn_body(q_ref, k_ref, v_ref, o_ref):
    i = pl.program_id(1)
    q = q_ref[0]
    scale = 1.0 / math.sqrt(HD)

    def full_step(kb, carry):
        m, l, acc = carry
        kblk = k_ref[0, pl.ds(kb * KBLK, KBLK), :]
        vblk = v_ref[0, pl.ds(kb * KBLK, KBLK), :]
        s = lax.dot_general(q, kblk, (((1,), (1,)), ((), ())),
                            preferred_element_type=jnp.float32) * scale
        mn = jnp.maximum(m, jnp.max(s, axis=-1, keepdims=True))
        p = jnp.exp(s - mn)
        corr = jnp.exp(m - mn)
        l = l * corr + jnp.sum(p, axis=-1, keepdims=True)
        acc = acc * corr + jnp.dot(p, vblk, preferred_element_type=jnp.float32)
        return mn, l, acc

    m0 = jnp.full((QBLK, 1), -1e30, jnp.float32)
    l0 = jnp.zeros((QBLK, 1), jnp.float32)
    a0 = jnp.zeros((QBLK, HD), jnp.float32)
    m, l, acc = lax.fori_loop(0, i, full_step, (m0, l0, a0))

    kblk = k_ref[0, pl.ds(i * KBLK, KBLK), :]
    vblk = v_ref[0, pl.ds(i * KBLK, KBLK), :]
    s = lax.dot_general(q, kblk, (((1,), (1,)), ((), ())),
                        preferred_element_type=jnp.float32) * scale
    tri = (lax.broadcasted_iota(jnp.int32, (QBLK, KBLK), 0)
           >= lax.broadcasted_iota(jnp.int32, (QBLK, KBLK), 1))
    s = jnp.where(tri, s, -1e30)
    mn = jnp.maximum(m, jnp.max(s, axis=-1, keepdims=True))
    p = jnp.exp(s - mn)
    corr = jnp.exp(m - mn)
    l = l * corr + jnp.sum(p, axis=-1, keepdims=True)
    acc = acc * corr + jnp.dot(p, vblk, preferred_element_type=jnp.float32)
    o_ref[0] = acc / l


def _attention(q3, k3, v3):
    return pl.pallas_call(
        _attn_body,
        grid=(H, NQB),
        in_specs=[
            pl.BlockSpec((1, QBLK, HD), lambda h, i: (h, i, 0)),
            pl.BlockSpec((1, S, HD), lambda h, i: (h // GRP, 0, 0)),
            pl.BlockSpec((1, S, HD), lambda h, i: (h // GRP, 0, 0)),
        ],
        out_specs=pl.BlockSpec((1, QBLK, HD), lambda h, i: (h, i, 0)),
        out_shape=jax.ShapeDtypeStruct((H, S, HD), jnp.float32),
    )(q3, k3, v3)


def _post_body(o_ref, wo_ref, x_ref, s2_ref, rw_ref, rb_ref,
               x1_ref, h2_ref, rt_ref):
    o_flat = jnp.concatenate([o_ref[hh] for hh in range(H)], axis=-1)
    x1 = x_ref[...] + jnp.dot(o_flat, wo_ref[...],
                              preferred_element_type=jnp.float32)
    x1_ref[...] = x1
    h2 = _rmsn(x1, s2_ref[...])
    h2_ref[...] = jnp.concatenate(
        [h2, jnp.zeros((SBLK, DP - D), jnp.float32)], axis=-1)
    logits = (jnp.dot(h2, rw_ref[...], preferred_element_type=jnp.float32)
              + rb_ref[...])
    eio = lax.broadcasted_iota(jnp.int32, (SBLK, E), 1)
    m1 = jnp.max(logits, axis=-1, keepdims=True)
    i1 = jnp.min(jnp.where(logits == m1, eio, E), axis=-1, keepdims=True)
    l2 = jnp.where(eio == i1, -jnp.inf, logits)
    m2 = jnp.max(l2, axis=-1, keepdims=True)
    i2 = jnp.min(jnp.where(l2 == m2, eio, E), axis=-1, keepdims=True)
    g0 = 1.0 / (1.0 + jnp.exp(m2 - m1))
    g1 = 1.0 - g0
    z = jnp.zeros((SBLK, E - 4), jnp.float32)
    rt_ref[...] = jnp.concatenate(
        [i1.astype(jnp.float32), i2.astype(jnp.float32), g0, g1, z],
        axis=-1)


def _post_attn(o3, wo, x2, s2, rw, rb):
    return pl.pallas_call(
        _post_body,
        grid=(NSB,),
        in_specs=[
            pl.BlockSpec((H, SBLK, HD), lambda i: (0, i, 0)),
            pl.BlockSpec((H * HD, D), lambda i: (0, 0)),
            pl.BlockSpec((SBLK, D), lambda i: (i, 0)),
            pl.BlockSpec((1, D), lambda i: (0, 0)),
            pl.BlockSpec((D, E), lambda i: (0, 0)),
            pl.BlockSpec((1, E), lambda i: (0, 0)),
        ],
        out_specs=[
            pl.BlockSpec((SBLK, D), lambda i: (i, 0)),
            pl.BlockSpec((SBLK, DP), lambda i: (i, 0)),
            pl.BlockSpec((SBLK, E), lambda i: (i, 0)),
        ],
        out_shape=[
            jax.ShapeDtypeStruct((S, D), jnp.float32),
            jax.ShapeDtypeStruct((S, DP), jnp.float32),
            jax.ShapeDtypeStruct((S, E), jnp.float32),
        ],
    )(o3, wo, x2, s2, rw, rb)


def _excl_cumsum_rows(x):
    acc = x
    sh = 1
    while sh < S:
        acc = acc + jnp.concatenate(
            [jnp.zeros((sh, E), x.dtype), acc[:S - sh, :]], axis=0)
        sh *= 2
    return acc - x


def _excl_cumsum_lanes(x):
    acc = x
    for sh in (1, 2, 4):
        acc = acc + jnp.concatenate(
            [jnp.zeros((1, sh), x.dtype), acc[:, :E - sh]], axis=1)
    return acc - x


def _route_body(rt_ref, pos_ref, binfo_ref):
    rt = rt_ref[...]
    e0 = rt[:, 0:1].astype(jnp.int32)
    e1 = rt[:, 1:2].astype(jnp.int32)
    eio = lax.broadcasted_iota(jnp.int32, (S, E), 1)
    oh0 = (e0 == eio).astype(jnp.float32)
    oh1 = (e1 == eio).astype(jnp.float32)
    c0 = _excl_cumsum_rows(oh0)
    c1 = _excl_cumsum_rows(oh1)
    tot0 = jnp.sum(oh0, axis=0, keepdims=True)
    sizes = tot0 + jnp.sum(oh1, axis=0, keepdims=True)
    padded = jnp.floor((sizes + (BLK_M - 1)) / BLK_M) * BLK_M
    poff = _excl_cumsum_lanes(padded)
    pos0 = jnp.sum(oh0 * (poff + c0), axis=-1, keepdims=True)
    pos1 = jnp.sum(oh1 * (poff + tot0 + c1), axis=-1, keepdims=True)
    zs = jnp.zeros((S, E - 2), jnp.float32)
    pos_ref[...] = jnp.concatenate([pos0, pos1, zs], axis=-1).astype(jnp.int32)

    bstart = poff / BLK_M
    pb = padded / BLK_M
    nused = jnp.sum(pb)
    eiof = lax.broadcasted_iota(jnp.int32, (1, E), 1).astype(jnp.float32)
    belast = jnp.max(jnp.where(pb > 0, eiof, 0.0))
    bio = lax.broadcasted_iota(jnp.int32, (1, 128), 1).astype(jnp.float32)
    bi = jnp.zeros((1, 128), jnp.float32)
    for e in range(E):
        m = (bio >= bstart[0, e]) & (bio < bstart[0, e] + pb[0, e])
        bi = jnp.where(m, float(e), bi)
    bi = jnp.where((bio >= nused) & (bio < float(NB)), belast, bi)
    bi = jnp.where(bio == 32.0, nused, bi)
    binfo_ref[...] = bi.astype(jnp.int32)


def _route_plan(rt):
    return pl.pallas_call(
        _route_body,
        out_shape=[
            jax.ShapeDtypeStruct((S, E), jnp.int32),
            jax.ShapeDtypeStruct((1, 128), jnp.int32),
        ],
    )(rt)


_APW = TOPK * S // NW


@functools.lru_cache(maxsize=None)
def _sc_mesh():
    return plsc.VectorSubcoreMesh(core_axis_name="c", subcore_axis_name="s")


@functools.lru_cache(maxsize=None)
def _build_sc_dispatch():
    @functools.partial(
        pl.kernel,
        mesh=_sc_mesh(),
        out_type=jax.ShapeDtypeStruct((NBUF, DP), jnp.float32),
        scratch_types=[
            pltpu.VMEM((_APW,), jnp.int32),
            pltpu.VMEM((_APW, DP), jnp.float32),
            pltpu.SemaphoreType.DMA,
        ],
    )
    def dispatch(pos_hbm, h2_hbm, xs_hbm, idx_v, rows_v, sem):
        wid = lax.axis_index("s") * 2 + lax.axis_index("c")
        base = wid * _APW
        tbase = lax.rem(base, S)
        pltpu.sync_copy(pos_hbm.at[pl.ds(base, _APW)], idx_v)
        pltpu.sync_copy(h2_hbm.at[pl.ds(tbase, _APW)], rows_v)
        pltpu.async_copy(rows_v, xs_hbm.at[idx_v], sem).wait()

    return dispatch


def _sc_dispatch(pos_flat, h2):
    return _build_sc_dispatch()(pos_flat, h2)


_TPW = S // NW
_CH = 32


@functools.lru_cache(maxsize=None)
def _build_sc_combine():
    @functools.partial(
        pl.kernel,
        mesh=_sc_mesh(),
        out_type=(
            jax.ShapeDtypeStruct((S, DP), jnp.float32),
            jax.ShapeDtypeStruct((S, DP), jnp.float32),
        ),
        scratch_types=[
            pltpu.VMEM((_CH,), jnp.int32),
            pltpu.VMEM((_CH,), jnp.int32),
            pltpu.VMEM((_CH, DP), jnp.float32),
            pltpu.VMEM((_CH, DP), jnp.float32),
            pltpu.SemaphoreType.DMA,
            pltpu.SemaphoreType.DMA,
        ],
    )
    def combine(p0_hbm, p1_hbm, eo_hbm, r0_hbm, r1_hbm,
                i0_v, i1_v, r0_v, r1_v, s0, s1):
        wid = lax.axis_index("s") * 2 + lax.axis_index("c")
        for c in range(_TPW // _CH):
            base = wid * _TPW + c * _CH
            pltpu.sync_copy(p0_hbm.at[pl.ds(base, _CH)], i0_v)
            pltpu.sync_copy(p1_hbm.at[pl.ds(base, _CH)], i1_v)
            a = pltpu.async_copy(eo_hbm.at[i0_v], r0_v, s0)
            b = pltpu.async_copy(eo_hbm.at[i1_v], r1_v, s1)
            a.wait()
            b.wait()
            pltpu.sync_copy(r0_v, r0_hbm.at[pl.ds(base, _CH)])
            pltpu.sync_copy(r1_v, r1_hbm.at[pl.ds(base, _CH)])

    return combine


def _sc_combine(pos0, pos1, eo):
    return _build_sc_combine()(pos0, pos1, eo)


def _ffn_body(b_ref, xs_ref, w1_ref, b1_ref, w2_ref, b2_ref, eo_ref):
    i = pl.program_id(0)
    nused = b_ref[32]

    @pl.when(i < nused)
    def _():
        hid = (jnp.dot(xs_ref[...][:, :D], w1_ref[0],
                       preferred_element_type=jnp.float32) + b1_ref[0])
        act = jnp.square(jnp.maximum(hid, 0.0))
        contrib = (jnp.dot(act, w2_ref[0], preferred_element_type=jnp.float32)
                   + b2_ref[0])
        zpad = jnp.zeros((BLK_M, DP - D), jnp.float32)
        eo_ref[...] = jnp.concatenate([contrib, zpad], axis=-1)


def _expert_ffn(binfo, xs, w1, b1, w2, b2):
    grid_spec = pltpu.PrefetchScalarGridSpec(
        num_scalar_prefetch=1,
        grid=(NB,),
        in_specs=[
            pl.BlockSpec((BLK_M, DP), lambda i, b: (i, 0)),
            pl.BlockSpec((1, D, DFF), lambda i, b: (b[i], 0, 0)),
            pl.BlockSpec((1, 1, DFF), lambda i, b: (b[i], 0, 0)),
            pl.BlockSpec((1, DFF, D), lambda i, b: (b[i], 0, 0)),
            pl.BlockSpec((1, 1, D), lambda i, b: (b[i], 0, 0)),
        ],
        out_specs=pl.BlockSpec((BLK_M, DP), lambda i, b: (i, 0)),
    )
    return pl.pallas_call(
        _ffn_body,
        grid_spec=grid_spec,
        out_shape=jax.ShapeDtypeStruct((NBUF, DP), jnp.float32),
    )(binfo, xs, w1, b1.reshape(E, 1, DFF), w2, b2.reshape(E, 1, D))


def _comb_body(x1_ref, r0_ref, r1_ref, rt_ref, y_ref):
    g0 = rt_ref[:, 2:3]
    g1 = rt_ref[:, 3:4]
    y_ref[...] = (x1_ref[...] + g0 * r0_ref[...][:, :D]
                  + g1 * r1_ref[...][:, :D])


def _final_combine(x1, r0, r1, rt):
    return pl.pallas_call(
        _comb_body,
        grid=(NSB,),
        in_specs=[
            pl.BlockSpec((SBLK, D), lambda i: (i, 0)),
            pl.BlockSpec((SBLK, DP), lambda i: (i, 0)),
            pl.BlockSpec((SBLK, DP), lambda i: (i, 0)),
            pl.BlockSpec((SBLK, E), lambda i: (i, 0)),
        ],
        out_specs=pl.BlockSpec((SBLK, D), lambda i: (i, 0)),
        out_shape=jax.ShapeDtypeStruct((S, D), jnp.float32),
    )(x1, r0, r1, rt)


def kernel(x, rms1_scale, wq, wk, wv, wo, q_norm_scale, k_norm_scale,
           rms2_scale, router_w, router_b, w1, b1, w2, b2):
    x2 = x.reshape(S, D)
    s1 = rms1_scale.reshape(1, D)
    s2 = rms2_scale.reshape(1, D)
    rb = router_b.reshape(1, E)
    qsf = jnp.tile(q_norm_scale, H).reshape(1, H * HD)
    ksf = jnp.tile(k_norm_scale, KVH).reshape(1, KVH * HD)
    gq = jnp.repeat(jnp.eye(H, dtype=jnp.float32), HD, axis=0)
    gk = jnp.repeat(jnp.eye(KVH, dtype=jnp.float32), HD, axis=0)

    inv = 1.0 / (10000.0 ** (jnp.arange(HALF, dtype=jnp.float32) / HALF))
    ang = jnp.arange(S, dtype=jnp.float32)[:, None] * inv[None, :]
    cos = jnp.cos(ang)
    sin = jnp.sin(ang)
    cos2 = jnp.concatenate([cos, cos], axis=1)
    sin2 = jnp.concatenate([-sin, sin], axis=1)
    cfq = jnp.tile(cos2, (1, H))
    sfq = jnp.tile(sin2, (1, H))
    cfk = jnp.tile(cos2, (1, KVH))
    sfk = jnp.tile(sin2, (1, KVH))

    q3, k3, v3 = _qkv_proj(x2, s1, wq, wk, wv, qsf, ksf, gq, gk,
                           cfq, sfq, cfk, sfk)
    o3 = _attention(q3, k3, v3)
    x1, h2, rt = _post_attn(o3, wo, x2, s2, router_w, rb)

    pos, binfo = _route_plan(rt)
    pos_flat = jnp.concatenate([pos[:, 0], pos[:, 1]], axis=0)
    xs = _sc_dispatch(pos_flat, h2)
    eo = _expert_ffn(binfo.reshape(128), xs, w1, b1, w2, b2)
    r0, r1 = _sc_combine(pos[:, 0], pos[:, 1], eo)
    y = _final_combine(x1, r0, r1, rt)
    return y.reshape(1, S, D)


# speedup vs baseline: 1.6025x; 1.0926x over previous
import functools
import math

import jax
import jax.numpy as jnp
from jax import lax
from jax.experimental import pallas as pl
from jax.experimental.pallas import tpu as pltpu
from jax.experimental.pallas import tpu_sc as plsc

S, D = 2048, 672
H, KVH, HD = 12, 4, 56
E, DFF, TOPK = 8, 2048, 2
EPS = 1e-05
HALF = HD // 2
GRP = H // KVH

SBLK = 256
NSB = S // SBLK
KBLK = 256
BLK_M = 256
NBUF = TOPK * S + E * BLK_M
NB = NBUF // BLK_M
DCH = 512
NDC = DFF // DCH
NW = 32
DP = 768


def _rmsn(x, scale):
    return x * lax.rsqrt(jnp.mean(x * x, axis=-1, keepdims=True) + EPS) * scale


def _swap_halves(t):
    n, w = t.shape
    z = jnp.zeros((n, HALF), t.dtype)
    lo = jnp.concatenate([t[:, HALF:], z], axis=1)
    hi = jnp.concatenate([z, t[:, :w - HALF]], axis=1)
    lane = lax.broadcasted_iota(jnp.int32, t.shape, 1)
    return jnp.where(lane % HD < HALF, lo, hi)


def _head_rms(q, gmat, gmat_t, scale_f):
    gs = jnp.dot(q * q, gmat, preferred_element_type=jnp.float32)
    scl = lax.rsqrt(gs * (1.0 / HD) + EPS)
    return q * jnp.dot(scl, gmat_t, preferred_element_type=jnp.float32) * scale_f


def _qkv_body(x_ref, s1_ref, wq_ref, wk_ref, wv_ref, qs_ref, ks_ref,
              gq_ref, gqt_ref, gk_ref, gkt_ref, c2_ref, s2r_ref,
              qo_ref, ko_ref, vo_ref):
    h = _rmsn(x_ref[...], s1_ref[...])
    cfq = jnp.tile(c2_ref[...], (1, H))
    sfq = jnp.tile(s2r_ref[...], (1, H))
    cfk = jnp.tile(c2_ref[...], (1, KVH))
    sfk = jnp.tile(s2r_ref[...], (1, KVH))
    q = jnp.dot(h, wq_ref[...], preferred_element_type=jnp.float32)
    q = _head_rms(q, gq_ref[...], gqt_ref[...], qs_ref[...])
    qr = q * cfq + _swap_halves(q) * sfq
    for hh in range(H):
        qo_ref[hh] = qr[:, HD * hh:HD * (hh + 1)]
    k = jnp.dot(h, wk_ref[...], preferred_element_type=jnp.float32)
    k = _head_rms(k, gk_ref[...], gkt_ref[...], ks_ref[...])
    kr = k * cfk + _swap_halves(k) * sfk
    v = jnp.dot(h, wv_ref[...], preferred_element_type=jnp.float32)
    for hh in range(KVH):
        ko_ref[hh] = kr[:, HD * hh:HD * (hh + 1)]
        vo_ref[hh] = v[:, HD * hh:HD * (hh + 1)]


def _qkv_proj(x2, s1, wq, wk, wv, qsf, ksf, gq, gk, cos2, sin2):
    kvd = KVH * HD
    c0 = lambda i: (0, 0)
    return pl.pallas_call(
        _qkv_body,
        grid=(NSB,),
        in_specs=[
            pl.BlockSpec((SBLK, D), lambda i: (i, 0)),
            pl.BlockSpec((1, D), c0),
            pl.BlockSpec((D, H * HD), c0),
            pl.BlockSpec((D, kvd), c0),
            pl.BlockSpec((D, kvd), c0),
            pl.BlockSpec((1, H * HD), c0),
            pl.BlockSpec((1, kvd), c0),
            pl.BlockSpec((H * HD, H), c0),
            pl.BlockSpec((H, H * HD), c0),
            pl.BlockSpec((kvd, KVH), c0),
            pl.BlockSpec((KVH, kvd), c0),
            pl.BlockSpec((SBLK, HD), lambda i: (i, 0)),
            pl.BlockSpec((SBLK, HD), lambda i: (i, 0)),
        ],
        out_specs=[
            pl.BlockSpec((H, SBLK, HD), lambda i: (0, i, 0)),
            pl.BlockSpec((KVH, SBLK, HD), lambda i: (0, i, 0)),
            pl.BlockSpec((KVH, SBLK, HD), lambda i: (0, i, 0)),
        ],
        out_shape=[
            jax.ShapeDtypeStruct((H, S, HD), jnp.float32),
            jax.ShapeDtypeStruct((KVH, S, HD), jnp.float32),
            jax.ShapeDtypeStruct((KVH, S, HD), jnp.float32),
        ],
    )(x2, s1, wq, wk, wv, qsf, ksf, gq, gq.T, gk, gk.T, cos2, sin2)


QBLK = 512
KBLK = 512
NQB = S // QBLK


def _attn_body(q_ref, k_ref, v_ref, o_ref):
    i = pl.program_id(1)
    q = q_ref[0]
    scale = 1.0 / math.sqrt(HD)

    def full_step(kb, carry):
        m, l, acc = carry
        kblk = k_ref[0, pl.ds(kb * KBLK, KBLK), :]
        vblk = v_ref[0, pl.ds(kb * KBLK, KBLK), :]
        s = lax.dot_general(q, kblk, (((1,), (1,)), ((), ())),
                            preferred_element_type=jnp.float32) * scale
        mn = jnp.maximum(m, jnp.max(s, axis=-1, keepdims=True))
        p = jnp.exp(s - mn)
        corr = jnp.exp(m - mn)
        l = l * corr + jnp.sum(p, axis=-1, keepdims=True)
        acc = acc * corr + jnp.dot(p, vblk, preferred_element_type=jnp.float32)
        return mn, l, acc

    m0 = jnp.full((QBLK, 1), -1e30, jnp.float32)
    l0 = jnp.zeros((QBLK, 1), jnp.float32)
    a0 = jnp.zeros((QBLK, HD), jnp.float32)
    m, l, acc = lax.fori_loop(0, i, full_step, (m0, l0, a0))

    kblk = k_ref[0, pl.ds(i * KBLK, KBLK), :]
    vblk = v_ref[0, pl.ds(i * KBLK, KBLK), :]
    s = lax.dot_general(q, kblk, (((1,), (1,)), ((), ())),
                        preferred_element_type=jnp.float32) * scale
    tri = (lax.broadcasted_iota(jnp.int32, (QBLK, KBLK), 0)
           >= lax.broadcasted_iota(jnp.int32, (QBLK, KBLK), 1))
    s = jnp.where(tri, s, -1e30)
    mn = jnp.maximum(m, jnp.max(s, axis=-1, keepdims=True))
    p = jnp.exp(s - mn)
    corr = jnp.exp(m - mn)
    l = l * corr + jnp.sum(p, axis=-1, keepdims=True)
    acc = acc * corr + jnp.dot(p, vblk, preferred_element_type=jnp.float32)
    o_ref[0] = acc / l


def _attention(q3, k3, v3):
    return pl.pallas_call(
        _attn_body,
        grid=(H, NQB),
        in_specs=[
            pl.BlockSpec((1, QBLK, HD), lambda h, i: (h, i, 0)),
            pl.BlockSpec((1, S, HD), lambda h, i: (h // GRP, 0, 0)),
            pl.BlockSpec((1, S, HD), lambda h, i: (h // GRP, 0, 0)),
        ],
        out_specs=pl.BlockSpec((1, QBLK, HD), lambda h, i: (h, i, 0)),
        out_shape=jax.ShapeDtypeStruct((H, S, HD), jnp.float32),
    )(q3, k3, v3)


def _post_body(o_ref, wo_ref, x_ref, s2_ref, rw_ref, rb_ref,
               x1_ref, h2_ref, rt_ref):
    o_flat = jnp.concatenate([o_ref[hh] for hh in range(H)], axis=-1)
    x1 = x_ref[...] + jnp.dot(o_flat, wo_ref[...],
                              preferred_element_type=jnp.float32)
    x1_ref[...] = x1
    h2 = _rmsn(x1, s2_ref[...])
    h2_ref[...] = jnp.concatenate(
        [h2, jnp.zeros((SBLK, DP - D), jnp.float32)], axis=-1)
    logits = (jnp.dot(h2, rw_ref[...], preferred_element_type=jnp.float32)
              + rb_ref[...])
    eio = lax.broadcasted_iota(jnp.int32, (SBLK, E), 1)
    m1 = jnp.max(logits, axis=-1, keepdims=True)
    i1 = jnp.min(jnp.where(logits == m1, eio, E), axis=-1, keepdims=True)
    l2 = jnp.where(eio == i1, -jnp.inf, logits)
    m2 = jnp.max(l2, axis=-1, keepdims=True)
    i2 = jnp.min(jnp.where(l2 == m2, eio, E), axis=-1, keepdims=True)
    g0 = 1.0 / (1.0 + jnp.exp(m2 - m1))
    g1 = 1.0 - g0
    z = jnp.zeros((SBLK, E - 4), jnp.float32)
    rt_ref[...] = jnp.concatenate(
        [i1.astype(jnp.float32), i2.astype(jnp.float32), g0, g1, z],
        axis=-1)


def _post_attn(o3, wo, x2, s2, rw, rb):
    return pl.pallas_call(
        _post_body,
        grid=(NSB,),
        in_specs=[
            pl.BlockSpec((H, SBLK, HD), lambda i: (0, i, 0)),
            pl.BlockSpec((H * HD, D), lambda i: (0, 0)),
            pl.BlockSpec((SBLK, D), lambda i: (i, 0)),
            pl.BlockSpec((1, D), lambda i: (0, 0)),
            pl.BlockSpec((D, E), lambda i: (0, 0)),
            pl.BlockSpec((1, E), lambda i: (0, 0)),
        ],
        out_specs=[
            pl.BlockSpec((SBLK, D), lambda i: (i, 0)),
            pl.BlockSpec((SBLK, DP), lambda i: (i, 0)),
            pl.BlockSpec((SBLK, E), lambda i: (i, 0)),
        ],
        out_shape=[
            jax.ShapeDtypeStruct((S, D), jnp.float32),
            jax.ShapeDtypeStruct((S, DP), jnp.float32),
            jax.ShapeDtypeStruct((S, E), jnp.float32),
        ],
    )(o3, wo, x2, s2, rw, rb)


def _excl_cumsum_rows(x):
    acc = x
    sh = 1
    while sh < S:
        acc = acc + jnp.concatenate(
            [jnp.zeros((sh, E), x.dtype), acc[:S - sh, :]], axis=0)
        sh *= 2
    return acc - x


def _excl_cumsum_lanes(x):
    acc = x
    for sh in (1, 2, 4):
        acc = acc + jnp.concatenate(
            [jnp.zeros((1, sh), x.dtype), acc[:, :E - sh]], axis=1)
    return acc - x


def _route_body(rt_ref, pos_ref, binfo_ref):
    rt = rt_ref[...]
    e0 = rt[:, 0:1].astype(jnp.int32)
    e1 = rt[:, 1:2].astype(jnp.int32)
    eio = lax.broadcasted_iota(jnp.int32, (S, E), 1)
    oh0 = (e0 == eio).astype(jnp.float32)
    oh1 = (e1 == eio).astype(jnp.float32)
    c0 = _excl_cumsum_rows(oh0)
    c1 = _excl_cumsum_rows(oh1)
    tot0 = jnp.sum(oh0, axis=0, keepdims=True)
    sizes = tot0 + jnp.sum(oh1, axis=0, keepdims=True)
    padded = jnp.floor((sizes + (BLK_M - 1)) / BLK_M) * BLK_M
    poff = _excl_cumsum_lanes(padded)
    pos0 = jnp.sum(oh0 * (poff + c0), axis=-1, keepdims=True)
    pos1 = jnp.sum(oh1 * (poff + tot0 + c1), axis=-1, keepdims=True)
    zs = jnp.zeros((S, E - 2), jnp.float32)
    pos_ref[...] = jnp.concatenate([pos0, pos1, zs], axis=-1).astype(jnp.int32)

    bstart = poff / BLK_M
    pb = padded / BLK_M
    nused = jnp.sum(pb)
    eiof = lax.broadcasted_iota(jnp.int32, (1, E), 1).astype(jnp.float32)
    belast = jnp.max(jnp.where(pb > 0, eiof, 0.0))
    bio = lax.broadcasted_iota(jnp.int32, (1, 128), 1).astype(jnp.float32)
    bi = jnp.zeros((1, 128), jnp.float32)
    for e in range(E):
        m = (bio >= bstart[0, e]) & (bio < bstart[0, e] + pb[0, e])
        bi = jnp.where(m, float(e), bi)
    bi = jnp.where((bio >= nused) & (bio < float(NB)), belast, bi)
    bi = jnp.where(bio == 32.0, nused, bi)
    binfo_ref[...] = bi.astype(jnp.int32)


def _route_plan(rt):
    return pl.pallas_call(
        _route_body,
        out_shape=[
            jax.ShapeDtypeStruct((S, E), jnp.int32),
            jax.ShapeDtypeStruct((1, 128), jnp.int32),
        ],
    )(rt)


_APW = TOPK * S // NW


@functools.lru_cache(maxsize=None)
def _sc_mesh():
    return plsc.VectorSubcoreMesh(core_axis_name="c", subcore_axis_name="s")


@functools.lru_cache(maxsize=None)
def _build_sc_dispatch():
    @functools.partial(
        pl.kernel,
        mesh=_sc_mesh(),
        out_type=jax.ShapeDtypeStruct((NBUF, DP), jnp.float32),
        scratch_types=[
            pltpu.VMEM((_APW,), jnp.int32),
            pltpu.VMEM((_APW, DP), jnp.float32),
            pltpu.SemaphoreType.DMA,
        ],
    )
    def dispatch(pos_hbm, h2_hbm, xs_hbm, idx_v, rows_v, sem):
        wid = lax.axis_index("s") * 2 + lax.axis_index("c")
        base = wid * _APW
        tbase = lax.rem(base, S)
        pltpu.sync_copy(pos_hbm.at[pl.ds(base, _APW)], idx_v)
        pltpu.sync_copy(h2_hbm.at[pl.ds(tbase, _APW)], rows_v)
        pltpu.async_copy(rows_v, xs_hbm.at[idx_v], sem).wait()

    return dispatch


def _sc_dispatch(pos_flat, h2):
    return _build_sc_dispatch()(pos_flat, h2)


_TPW = S // NW
_CH = 32


@functools.lru_cache(maxsize=None)
def _build_sc_combine():
    @functools.partial(
        pl.kernel,
        mesh=_sc_mesh(),
        out_type=(
            jax.ShapeDtypeStruct((S, DP), jnp.float32),
            jax.ShapeDtypeStruct((S, DP), jnp.float32),
        ),
        scratch_types=[
            pltpu.VMEM((_CH,), jnp.int32),
            pltpu.VMEM((_CH,), jnp.int32),
            pltpu.VMEM((_CH, DP), jnp.float32),
            pltpu.VMEM((_CH, DP), jnp.float32),
            pltpu.SemaphoreType.DMA,
            pltpu.SemaphoreType.DMA,
        ],
    )
    def combine(p0_hbm, p1_hbm, eo_hbm, r0_hbm, r1_hbm,
                i0_v, i1_v, r0_v, r1_v, s0, s1):
        wid = lax.axis_index("s") * 2 + lax.axis_index("c")
        for c in range(_TPW // _CH):
            base = wid * _TPW + c * _CH
            pltpu.sync_copy(p0_hbm.at[pl.ds(base, _CH)], i0_v)
            pltpu.sync_copy(p1_hbm.at[pl.ds(base, _CH)], i1_v)
            a = pltpu.async_copy(eo_hbm.at[i0_v], r0_v, s0)
            b = pltpu.async_copy(eo_hbm.at[i1_v], r1_v, s1)
            a.wait()
            b.wait()
            pltpu.sync_copy(r0_v, r0_hbm.at[pl.ds(base, _CH)])
            pltpu.sync_copy(r1_v, r1_hbm.at[pl.ds(base, _CH)])

    return combine


def _sc_combine(pos0, pos1, eo):
    return _build_sc_combine()(pos0, pos1, eo)


def _ffn_body(b_ref, xs_ref, w1_ref, b1_ref, w2_ref, b2_ref, eo_ref):
    i = pl.program_id(0)
    nused = b_ref[32]

    @pl.when(i < nused)
    def _():
        hid = (jnp.dot(xs_ref[...][:, :D], w1_ref[...],
                       preferred_element_type=jnp.float32) + b1_ref[0])
        act = jnp.square(jnp.maximum(hid, 0.0))
        contrib = (jnp.dot(act, w2_ref[...], preferred_element_type=jnp.float32)
                   + b2_ref[0])
        zpad = jnp.zeros((BLK_M, DP - D), jnp.float32)
        eo_ref[...] = jnp.concatenate([contrib, zpad], axis=-1)


def _expert_ffn(binfo, xs, w1, b1, w2, b2):
    grid_spec = pltpu.PrefetchScalarGridSpec(
        num_scalar_prefetch=1,
        grid=(NB,),
        in_specs=[
            pl.BlockSpec((BLK_M, DP), lambda i, b: (i, 0)),
            pl.BlockSpec((D, DFF), lambda i, b: (b[i], 0)),
            pl.BlockSpec((1, 1, DFF), lambda i, b: (b[i], 0, 0)),
            pl.BlockSpec((DFF, D), lambda i, b: (b[i], 0)),
            pl.BlockSpec((1, 1, D), lambda i, b: (b[i], 0, 0)),
        ],
        out_specs=pl.BlockSpec((BLK_M, DP), lambda i, b: (i, 0)),
    )
    return pl.pallas_call(
        _ffn_body,
        grid_spec=grid_spec,
        out_shape=jax.ShapeDtypeStruct((NBUF, DP), jnp.float32),
    )(binfo, xs, w1.reshape(E * D, DFF), b1.reshape(E, 1, DFF),
      w2.reshape(E * DFF, D), b2.reshape(E, 1, D))


def _comb_body(x1_ref, r0_ref, r1_ref, rt_ref, y_ref):
    g0 = rt_ref[:, 2:3]
    g1 = rt_ref[:, 3:4]
    y_ref[...] = (x1_ref[...] + g0 * r0_ref[...][:, :D]
                  + g1 * r1_ref[...][:, :D])


def _final_combine(x1, r0, r1, rt):
    return pl.pallas_call(
        _comb_body,
        grid=(NSB,),
        in_specs=[
            pl.BlockSpec((SBLK, D), lambda i: (i, 0)),
            pl.BlockSpec((SBLK, DP), lambda i: (i, 0)),
            pl.BlockSpec((SBLK, DP), lambda i: (i, 0)),
            pl.BlockSpec((SBLK, E), lambda i: (i, 0)),
        ],
        out_specs=pl.BlockSpec((SBLK, D), lambda i: (i, 0)),
        out_shape=jax.ShapeDtypeStruct((S, D), jnp.float32),
    )(x1, r0, r1, rt)


def kernel(x, rms1_scale, wq, wk, wv, wo, q_norm_scale, k_norm_scale,
           rms2_scale, router_w, router_b, w1, b1, w2, b2):
    x2 = x.reshape(S, D)
    s1 = rms1_scale.reshape(1, D)
    s2 = rms2_scale.reshape(1, D)
    rb = router_b.reshape(1, E)
    qsf = jnp.tile(q_norm_scale, H).reshape(1, H * HD)
    ksf = jnp.tile(k_norm_scale, KVH).reshape(1, KVH * HD)
    gq = jnp.repeat(jnp.eye(H, dtype=jnp.float32), HD, axis=0)
    gk = jnp.repeat(jnp.eye(KVH, dtype=jnp.float32), HD, axis=0)

    inv = 1.0 / (10000.0 ** (jnp.arange(HALF, dtype=jnp.float32) / HALF))
    ang = jnp.arange(S, dtype=jnp.float32)[:, None] * inv[None, :]
    cos = jnp.cos(ang)
    sin = jnp.sin(ang)
    cos2 = jnp.concatenate([cos, cos], axis=1)
    sin2 = jnp.concatenate([-sin, sin], axis=1)

    q3, k3, v3 = _qkv_proj(x2, s1, wq, wk, wv, qsf, ksf, gq, gk, cos2, sin2)
    o3 = _attention(q3, k3, v3)
    x1, h2, rt = _post_attn(o3, wo, x2, s2, router_w, rb)

    pos, binfo = _route_plan(rt)
    pos_flat = jnp.concatenate([pos[:, 0], pos[:, 1]], axis=0)
    xs = _sc_dispatch(pos_flat, h2)
    eo = _expert_ffn(binfo.reshape(128), xs, w1, b1, w2, b2)
    r0, r1 = _sc_combine(pos[:, 0], pos[:, 1], eo)
    y = _final_combine(x1, r0, r1, rt)
    return y.reshape(1, S, D)


# speedup vs baseline: 1.6172x; 1.0092x over previous
import functools
import math

import jax
import jax.numpy as jnp
from jax import lax
from jax.experimental import pallas as pl
from jax.experimental.pallas import tpu as pltpu
from jax.experimental.pallas import tpu_sc as plsc

S, D = 2048, 672
H, KVH, HD = 12, 4, 56
E, DFF, TOPK = 8, 2048, 2
EPS = 1e-05
HALF = HD // 2
GRP = H // KVH

SBLK = 256
NSB = S // SBLK
KBLK = 256
BLK_M = 256
NBUF = TOPK * S + E * BLK_M
NB = NBUF // BLK_M
DCH = 512
NDC = DFF // DCH
NW = 32
DP = 768


def _rmsn(x, scale):
    return x * lax.rsqrt(jnp.mean(x * x, axis=-1, keepdims=True) + EPS) * scale


def _swap_halves(t):
    n, w = t.shape
    z = jnp.zeros((n, HALF), t.dtype)
    lo = jnp.concatenate([t[:, HALF:], z], axis=1)
    hi = jnp.concatenate([z, t[:, :w - HALF]], axis=1)
    lane = lax.broadcasted_iota(jnp.int32, t.shape, 1)
    return jnp.where(lane % HD < HALF, lo, hi)


def _head_rms(q, gmat, gmat_t, scale_f):
    gs = jnp.dot(q * q, gmat, preferred_element_type=jnp.float32)
    scl = lax.rsqrt(gs * (1.0 / HD) + EPS)
    return q * jnp.dot(scl, gmat_t, preferred_element_type=jnp.float32) * scale_f


def _qkv_body(x_ref, s1_ref, wq_ref, wk_ref, wv_ref, qs_ref, ks_ref,
              gq_ref, gqt_ref, gk_ref, gkt_ref, c2_ref, s2r_ref,
              qo_ref, ko_ref, vo_ref):
    h = _rmsn(x_ref[...], s1_ref[...])
    cfq = jnp.tile(c2_ref[...], (1, H))
    sfq = jnp.tile(s2r_ref[...], (1, H))
    cfk = jnp.tile(c2_ref[...], (1, KVH))
    sfk = jnp.tile(s2r_ref[...], (1, KVH))
    q = jnp.dot(h, wq_ref[...], preferred_element_type=jnp.float32)
    q = _head_rms(q, gq_ref[...], gqt_ref[...], qs_ref[...])
    qr = (q * cfq + _swap_halves(q) * sfq).astype(jnp.bfloat16)
    for hh in range(H):
        qo_ref[hh] = qr[:, HD * hh:HD * (hh + 1)]
    k = jnp.dot(h, wk_ref[...], preferred_element_type=jnp.float32)
    k = _head_rms(k, gk_ref[...], gkt_ref[...], ks_ref[...])
    kr = (k * cfk + _swap_halves(k) * sfk).astype(jnp.bfloat16)
    v = jnp.dot(h, wv_ref[...], preferred_element_type=jnp.float32)
    vb = v.astype(jnp.bfloat16)
    for hh in range(KVH):
        ko_ref[hh] = kr[:, HD * hh:HD * (hh + 1)]
        vo_ref[hh] = vb[:, HD * hh:HD * (hh + 1)]


def _qkv_proj(x2, s1, wq, wk, wv, qsf, ksf, gq, gk, cos2, sin2):
    kvd = KVH * HD
    c0 = lambda i: (0, 0)
    return pl.pallas_call(
        _qkv_body,
        grid=(NSB,),
        in_specs=[
            pl.BlockSpec((SBLK, D), lambda i: (i, 0)),
            pl.BlockSpec((1, D), c0),
            pl.BlockSpec((D, H * HD), c0),
            pl.BlockSpec((D, kvd), c0),
            pl.BlockSpec((D, kvd), c0),
            pl.BlockSpec((1, H * HD), c0),
            pl.BlockSpec((1, kvd), c0),
            pl.BlockSpec((H * HD, H), c0),
            pl.BlockSpec((H, H * HD), c0),
            pl.BlockSpec((kvd, KVH), c0),
            pl.BlockSpec((KVH, kvd), c0),
            pl.BlockSpec((SBLK, HD), lambda i: (i, 0)),
            pl.BlockSpec((SBLK, HD), lambda i: (i, 0)),
        ],
        out_specs=[
            pl.BlockSpec((H, SBLK, HD), lambda i: (0, i, 0)),
            pl.BlockSpec((KVH, SBLK, HD), lambda i: (0, i, 0)),
            pl.BlockSpec((KVH, SBLK, HD), lambda i: (0, i, 0)),
        ],
        out_shape=[
            jax.ShapeDtypeStruct((H, S, HD), jnp.bfloat16),
            jax.ShapeDtypeStruct((KVH, S, HD), jnp.bfloat16),
            jax.ShapeDtypeStruct((KVH, S, HD), jnp.bfloat16),
        ],
    )(x2, s1, wq, wk, wv, qsf, ksf, gq, gq.T, gk, gk.T, cos2, sin2)


QBLK = 512
KBLK = 512
NQB = S // QBLK


def _attn_body(q_ref, k_ref, v_ref, o_ref):
    i = pl.program_id(1)
    q = (q_ref[0].astype(jnp.float32) * (1.0 / math.sqrt(HD))).astype(
        jnp.bfloat16)

    def full_step(kb, carry):
        m, l, acc = carry
        kblk = k_ref[0, pl.ds(kb * KBLK, KBLK), :]
        vblk = v_ref[0, pl.ds(kb * KBLK, KBLK), :]
        s = lax.dot_general(q, kblk, (((1,), (1,)), ((), ())),
                            preferred_element_type=jnp.float32)
        mn = jnp.maximum(m, jnp.max(s, axis=-1, keepdims=True))
        p = jnp.exp(s - mn)
        corr = jnp.exp(m - mn)
        l = l * corr + jnp.sum(p, axis=-1, keepdims=True)
        acc = acc * corr + jnp.dot(p.astype(jnp.bfloat16), vblk,
                                   preferred_element_type=jnp.float32)
        return mn, l, acc

    m0 = jnp.full((QBLK, 1), -1e30, jnp.float32)
    l0 = jnp.zeros((QBLK, 1), jnp.float32)
    a0 = jnp.zeros((QBLK, HD), jnp.float32)
    m, l, acc = lax.fori_loop(0, i, full_step, (m0, l0, a0))

    kblk = k_ref[0, pl.ds(i * KBLK, KBLK), :]
    vblk = v_ref[0, pl.ds(i * KBLK, KBLK), :]
    s = lax.dot_general(q, kblk, (((1,), (1,)), ((), ())),
                        preferred_element_type=jnp.float32)
    tri = (lax.broadcasted_iota(jnp.int32, (QBLK, KBLK), 0)
           >= lax.broadcasted_iota(jnp.int32, (QBLK, KBLK), 1))
    s = jnp.where(tri, s, -1e30)
    mn = jnp.maximum(m, jnp.max(s, axis=-1, keepdims=True))
    p = jnp.exp(s - mn)
    corr = jnp.exp(m - mn)
    l = l * corr + jnp.sum(p, axis=-1, keepdims=True)
    acc = acc * corr + jnp.dot(p.astype(jnp.bfloat16), vblk,
                               preferred_element_type=jnp.float32)
    o_ref[0] = acc / l


def _attention(q3, k3, v3):
    return pl.pallas_call(
        _attn_body,
        grid=(H, NQB),
        in_specs=[
            pl.BlockSpec((1, QBLK, HD), lambda h, i: (h, i, 0)),
            pl.BlockSpec((1, S, HD), lambda h, i: (h // GRP, 0, 0)),
            pl.BlockSpec((1, S, HD), lambda h, i: (h // GRP, 0, 0)),
        ],
        out_specs=pl.BlockSpec((1, QBLK, HD), lambda h, i: (h, i, 0)),
        out_shape=jax.ShapeDtypeStruct((H, S, HD), jnp.float32),
    )(q3, k3, v3)


def _post_body(o_ref, wo_ref, x_ref, s2_ref, rw_ref, rb_ref,
               x1_ref, h2_ref, rt_ref):
    o_flat = jnp.concatenate([o_ref[hh] for hh in range(H)], axis=-1)
    x1 = x_ref[...] + jnp.dot(o_flat, wo_ref[...],
                              preferred_element_type=jnp.float32)
    x1_ref[...] = x1
    h2 = _rmsn(x1, s2_ref[...])
    h2_ref[...] = jnp.concatenate(
        [h2, jnp.zeros((SBLK, DP - D), jnp.float32)], axis=-1)
    logits = (jnp.dot(h2, rw_ref[...], preferred_element_type=jnp.float32)
              + rb_ref[...])
    eio = lax.broadcasted_iota(jnp.int32, (SBLK, E), 1)
    m1 = jnp.max(logits, axis=-1, keepdims=True)
    i1 = jnp.min(jnp.where(logits == m1, eio, E), axis=-1, keepdims=True)
    l2 = jnp.where(eio == i1, -jnp.inf, logits)
    m2 = jnp.max(l2, axis=-1, keepdims=True)
    i2 = jnp.min(jnp.where(l2 == m2, eio, E), axis=-1, keepdims=True)
    g0 = 1.0 / (1.0 + jnp.exp(m2 - m1))
    g1 = 1.0 - g0
    z = jnp.zeros((SBLK, E - 4), jnp.float32)
    rt_ref[...] = jnp.concatenate(
        [i1.astype(jnp.float32), i2.astype(jnp.float32), g0, g1, z],
        axis=-1)


def _post_attn(o3, wo, x2, s2, rw, rb):
    return pl.pallas_call(
        _post_body,
        grid=(NSB,),
        in_specs=[
            pl.BlockSpec((H, SBLK, HD), lambda i: (0, i, 0)),
            pl.BlockSpec((H * HD, D), lambda i: (0, 0)),
            pl.BlockSpec((SBLK, D), lambda i: (i, 0)),
            pl.BlockSpec((1, D), lambda i: (0, 0)),
            pl.BlockSpec((D, E), lambda i: (0, 0)),
            pl.BlockSpec((1, E), lambda i: (0, 0)),
        ],
        out_specs=[
            pl.BlockSpec((SBLK, D), lambda i: (i, 0)),
            pl.BlockSpec((SBLK, DP), lambda i: (i, 0)),
            pl.BlockSpec((SBLK, E), lambda i: (i, 0)),
        ],
        out_shape=[
            jax.ShapeDtypeStruct((S, D), jnp.float32),
            jax.ShapeDtypeStruct((S, DP), jnp.float32),
            jax.ShapeDtypeStruct((S, E), jnp.float32),
        ],
    )(o3, wo, x2, s2, rw, rb)


def _excl_cumsum_rows(x):
    acc = x
    sh = 1
    while sh < S:
        acc = acc + jnp.concatenate(
            [jnp.zeros((sh, E), x.dtype), acc[:S - sh, :]], axis=0)
        sh *= 2
    return acc - x


def _excl_cumsum_lanes(x):
    acc = x
    for sh in (1, 2, 4):
        acc = acc + jnp.concatenate(
            [jnp.zeros((1, sh), x.dtype), acc[:, :E - sh]], axis=1)
    return acc - x


def _route_body(rt_ref, pos_ref, binfo_ref):
    rt = rt_ref[...]
    e0 = rt[:, 0:1].astype(jnp.int32)
    e1 = rt[:, 1:2].astype(jnp.int32)
    eio = lax.broadcasted_iota(jnp.int32, (S, E), 1)
    oh0 = (e0 == eio).astype(jnp.float32)
    oh1 = (e1 == eio).astype(jnp.float32)
    c0 = _excl_cumsum_rows(oh0)
    c1 = _excl_cumsum_rows(oh1)
    tot0 = jnp.sum(oh0, axis=0, keepdims=True)
    sizes = tot0 + jnp.sum(oh1, axis=0, keepdims=True)
    padded = jnp.floor((sizes + (BLK_M - 1)) / BLK_M) * BLK_M
    poff = _excl_cumsum_lanes(padded)
    pos0 = jnp.sum(oh0 * (poff + c0), axis=-1, keepdims=True)
    pos1 = jnp.sum(oh1 * (poff + tot0 + c1), axis=-1, keepdims=True)
    zs = jnp.zeros((S, E - 2), jnp.float32)
    pos_ref[...] = jnp.concatenate([pos0, pos1, zs], axis=-1).astype(jnp.int32)

    bstart = poff / BLK_M
    pb = padded / BLK_M
    nused = jnp.sum(pb)
    eiof = lax.broadcasted_iota(jnp.int32, (1, E), 1).astype(jnp.float32)
    belast = jnp.max(jnp.where(pb > 0, eiof, 0.0))
    bio = lax.broadcasted_iota(jnp.int32, (1, 128), 1).astype(jnp.float32)
    bi = jnp.zeros((1, 128), jnp.float32)
    for e in range(E):
        m = (bio >= bstart[0, e]) & (bio < bstart[0, e] + pb[0, e])
        bi = jnp.where(m, float(e), bi)
    bi = jnp.where((bio >= nused) & (bio < float(NB)), belast, bi)
    bi = jnp.where(bio == 32.0, nused, bi)
    binfo_ref[...] = bi.astype(jnp.int32)


def _route_plan(rt):
    return pl.pallas_call(
        _route_body,
        out_shape=[
            jax.ShapeDtypeStruct((S, E), jnp.int32),
            jax.ShapeDtypeStruct((1, 128), jnp.int32),
        ],
    )(rt)


_APW = TOPK * S // NW


@functools.lru_cache(maxsize=None)
def _sc_mesh():
    return plsc.VectorSubcoreMesh(core_axis_name="c", subcore_axis_name="s")


@functools.lru_cache(maxsize=None)
def _build_sc_dispatch():
    @functools.partial(
        pl.kernel,
        mesh=_sc_mesh(),
        out_type=jax.ShapeDtypeStruct((NBUF, DP), jnp.float32),
        scratch_types=[
            pltpu.VMEM((_APW,), jnp.int32),
            pltpu.VMEM((_APW, DP), jnp.float32),
            pltpu.SemaphoreType.DMA,
        ],
    )
    def dispatch(pos_hbm, h2_hbm, xs_hbm, idx_v, rows_v, sem):
        wid = lax.axis_index("s") * 2 + lax.axis_index("c")
        base = wid * _APW
        tbase = lax.rem(base, S)
        pltpu.sync_copy(pos_hbm.at[pl.ds(base, _APW)], idx_v)
        pltpu.sync_copy(h2_hbm.at[pl.ds(tbase, _APW)], rows_v)
        pltpu.async_copy(rows_v, xs_hbm.at[idx_v], sem).wait()

    return dispatch


def _sc_dispatch(pos_flat, h2):
    return _build_sc_dispatch()(pos_flat, h2)


_TPW = S // NW
_CH = 32


@functools.lru_cache(maxsize=None)
def _build_sc_combine():
    @functools.partial(
        pl.kernel,
        mesh=_sc_mesh(),
        out_type=(
            jax.ShapeDtypeStruct((S, DP), jnp.float32),
            jax.ShapeDtypeStruct((S, DP), jnp.float32),
        ),
        scratch_types=[
            pltpu.VMEM((_CH,), jnp.int32),
            pltpu.VMEM((_CH,), jnp.int32),
            pltpu.VMEM((_CH, DP), jnp.float32),
            pltpu.VMEM((_CH, DP), jnp.float32),
            pltpu.SemaphoreType.DMA,
            pltpu.SemaphoreType.DMA,
        ],
    )
    def combine(p0_hbm, p1_hbm, eo_hbm, r0_hbm, r1_hbm,
                i0_v, i1_v, r0_v, r1_v, s0, s1):
        wid = lax.axis_index("s") * 2 + lax.axis_index("c")
        for c in range(_TPW // _CH):
            base = wid * _TPW + c * _CH
            pltpu.sync_copy(p0_hbm.at[pl.ds(base, _CH)], i0_v)
            pltpu.sync_copy(p1_hbm.at[pl.ds(base, _CH)], i1_v)
            a = pltpu.async_copy(eo_hbm.at[i0_v], r0_v, s0)
            b = pltpu.async_copy(eo_hbm.at[i1_v], r1_v, s1)
            a.wait()
            b.wait()
            pltpu.sync_copy(r0_v, r0_hbm.at[pl.ds(base, _CH)])
            pltpu.sync_copy(r1_v, r1_hbm.at[pl.ds(base, _CH)])

    return combine


def _sc_combine(pos0, pos1, eo):
    return _build_sc_combine()(pos0, pos1, eo)


def _ffn_body(b_ref, xs_ref, w1_ref, b1_ref, w2_ref, b2_ref, eo_ref):
    i = pl.program_id(0)
    nused = b_ref[32]

    @pl.when(i < nused)
    def _():
        hid = (jnp.dot(xs_ref[...][:, :D], w1_ref[...],
                       preferred_element_type=jnp.float32) + b1_ref[0])
        act = jnp.square(jnp.maximum(hid, 0.0))
        contrib = (jnp.dot(act, w2_ref[...], preferred_element_type=jnp.float32)
                   + b2_ref[0])
        zpad = jnp.zeros((BLK_M, DP - D), jnp.float32)
        eo_ref[...] = jnp.concatenate([contrib, zpad], axis=-1)


def _expert_ffn(binfo, xs, w1, b1, w2, b2):
    grid_spec = pltpu.PrefetchScalarGridSpec(
        num_scalar_prefetch=1,
        grid=(NB,),
        in_specs=[
            pl.BlockSpec((BLK_M, DP), lambda i, b: (i, 0)),
            pl.BlockSpec((D, DFF), lambda i, b: (b[i], 0)),
            pl.BlockSpec((1, 1, DFF), lambda i, b: (b[i], 0, 0)),
            pl.BlockSpec((DFF, D), lambda i, b: (b[i], 0)),
            pl.BlockSpec((1, 1, D), lambda i, b: (b[i], 0, 0)),
        ],
        out_specs=pl.BlockSpec((BLK_M, DP), lambda i, b: (i, 0)),
    )
    return pl.pallas_call(
        _ffn_body,
        grid_spec=grid_spec,
        out_shape=jax.ShapeDtypeStruct((NBUF, DP), jnp.float32),
    )(binfo, xs, w1.reshape(E * D, DFF), b1.reshape(E, 1, DFF),
      w2.reshape(E * DFF, D), b2.reshape(E, 1, D))


def _comb_body(x1_ref, r0_ref, r1_ref, rt_ref, y_ref):
    g0 = rt_ref[:, 2:3]
    g1 = rt_ref[:, 3:4]
    y_ref[...] = (x1_ref[...] + g0 * r0_ref[...][:, :D]
                  + g1 * r1_ref[...][:, :D])


def _final_combine(x1, r0, r1, rt):
    return pl.pallas_call(
        _comb_body,
        grid=(NSB,),
        in_specs=[
            pl.BlockSpec((SBLK, D), lambda i: (i, 0)),
            pl.BlockSpec((SBLK, DP), lambda i: (i, 0)),
            pl.BlockSpec((SBLK, DP), lambda i: (i, 0)),
            pl.BlockSpec((SBLK, E), lambda i: (i, 0)),
        ],
        out_specs=pl.BlockSpec((SBLK, D), lambda i: (i, 0)),
        out_shape=jax.ShapeDtypeStruct((S, D), jnp.float32),
    )(x1, r0, r1, rt)


def kernel(x, rms1_scale, wq, wk, wv, wo, q_norm_scale, k_norm_scale,
           rms2_scale, router_w, router_b, w1, b1, w2, b2):
    x2 = x.reshape(S, D)
    s1 = rms1_scale.reshape(1, D)
    s2 = rms2_scale.reshape(1, D)
    rb = router_b.reshape(1, E)
    qsf = jnp.tile(q_norm_scale, H).reshape(1, H * HD)
    ksf = jnp.tile(k_norm_scale, KVH).reshape(1, KVH * HD)
    gq = jnp.repeat(jnp.eye(H, dtype=jnp.float32), HD, axis=0)
    gk = jnp.repeat(jnp.eye(KVH, dtype=jnp.float32), HD, axis=0)

    inv = 1.0 / (10000.0 ** (jnp.arange(HALF, dtype=jnp.float32) / HALF))
    ang = jnp.arange(S, dtype=jnp.float32)[:, None] * inv[None, :]
    cos = jnp.cos(ang)
    sin = jnp.sin(ang)
    cos2 = jnp.concatenate([cos, cos], axis=1)
    sin2 = jnp.concatenate([-sin, sin], axis=1)

    q3, k3, v3 = _qkv_proj(x2, s1, wq, wk, wv, qsf, ksf, gq, gk, cos2, sin2)
    o3 = _attention(q3, k3, v3)
    x1, h2, rt = _post_attn(o3, wo, x2, s2, router_w, rb)

    pos, binfo = _route_plan(rt)
    pos_flat = jnp.concatenate([pos[:, 0], pos[:, 1]], axis=0)
    xs = _sc_dispatch(pos_flat, h2)
    eo = _expert_ffn(binfo.reshape(128), xs, w1, b1, w2, b2)
    r0, r1 = _sc_combine(pos[:, 0], pos[:, 1], eo)
    y = _final_combine(x1, r0, r1, rt)
    return y.reshape(1, S, D)


# speedup vs baseline: 1.6259x; 1.0054x over previous
import functools
import math

import jax
import jax.numpy as jnp
from jax import lax
from jax.experimental import pallas as pl
from jax.experimental.pallas import tpu as pltpu
from jax.experimental.pallas import tpu_sc as plsc

S, D = 2048, 672
H, KVH, HD = 12, 4, 56
E, DFF, TOPK = 8, 2048, 2
EPS = 1e-05
HALF = HD // 2
GRP = H // KVH

SBLK = 256
NSB = S // SBLK
KBLK = 256
BLK_M = 256
NBUF = TOPK * S + E * BLK_M
NB = NBUF // BLK_M
DCH = 512
NDC = DFF // DCH
NW = 32
DP = 768


def _rmsn(x, scale):
    return x * lax.rsqrt(jnp.mean(x * x, axis=-1, keepdims=True) + EPS) * scale


def _swap_halves(t):
    n, w = t.shape
    z = jnp.zeros((n, HALF), t.dtype)
    lo = jnp.concatenate([t[:, HALF:], z], axis=1)
    hi = jnp.concatenate([z, t[:, :w - HALF]], axis=1)
    lane = lax.broadcasted_iota(jnp.int32, t.shape, 1)
    return jnp.where(lane % HD < HALF, lo, hi)


def _head_rms(q, gmat, gmat_t, scale_f):
    gs = jnp.dot(q * q, gmat, preferred_element_type=jnp.float32)
    scl = lax.rsqrt(gs * (1.0 / HD) + EPS)
    return q * jnp.dot(scl, gmat_t, preferred_element_type=jnp.float32) * scale_f


def _qkv_body(x_ref, s1_ref, wq_ref, wk_ref, wv_ref, qs_ref, ks_ref,
              gq_ref, gqt_ref, gk_ref, gkt_ref, c2_ref, s2r_ref,
              qo_ref, ko_ref, vo_ref):
    h = _rmsn(x_ref[...], s1_ref[...])
    cfq = jnp.tile(c2_ref[...], (1, H))
    sfq = jnp.tile(s2r_ref[...], (1, H))
    cfk = jnp.tile(c2_ref[...], (1, KVH))
    sfk = jnp.tile(s2r_ref[...], (1, KVH))
    q = jnp.dot(h, wq_ref[...], preferred_element_type=jnp.float32)
    q = _head_rms(q, gq_ref[...], gqt_ref[...], qs_ref[...])
    qr = q * cfq + _swap_halves(q) * sfq
    for hh in range(H):
        qo_ref[hh] = qr[:, HD * hh:HD * (hh + 1)]
    k = jnp.dot(h, wk_ref[...], preferred_element_type=jnp.float32)
    k = _head_rms(k, gk_ref[...], gkt_ref[...], ks_ref[...])
    kr = k * cfk + _swap_halves(k) * sfk
    v = jnp.dot(h, wv_ref[...], preferred_element_type=jnp.float32)
    for hh in range(KVH):
        ko_ref[hh] = kr[:, HD * hh:HD * (hh + 1)]
        vo_ref[hh] = v[:, HD * hh:HD * (hh + 1)]


def _qkv_proj(x2, s1, wq, wk, wv, qsf, ksf, gq, gk, cos2, sin2):
    kvd = KVH * HD
    c0 = lambda i: (0, 0)
    return pl.pallas_call(
        _qkv_body,
        grid=(NSB,),
        in_specs=[
            pl.BlockSpec((SBLK, D), lambda i: (i, 0)),
            pl.BlockSpec((1, D), c0),
            pl.BlockSpec((D, H * HD), c0),
            pl.BlockSpec((D, kvd), c0),
            pl.BlockSpec((D, kvd), c0),
            pl.BlockSpec((1, H * HD), c0),
            pl.BlockSpec((1, kvd), c0),
            pl.BlockSpec((H * HD, H), c0),
            pl.BlockSpec((H, H * HD), c0),
            pl.BlockSpec((kvd, KVH), c0),
            pl.BlockSpec((KVH, kvd), c0),
            pl.BlockSpec((SBLK, HD), lambda i: (i, 0)),
            pl.BlockSpec((SBLK, HD), lambda i: (i, 0)),
        ],
        out_specs=[
            pl.BlockSpec((H, SBLK, HD), lambda i: (0, i, 0)),
            pl.BlockSpec((KVH, SBLK, HD), lambda i: (0, i, 0)),
            pl.BlockSpec((KVH, SBLK, HD), lambda i: (0, i, 0)),
        ],
        out_shape=[
            jax.ShapeDtypeStruct((H, S, HD), jnp.float32),
            jax.ShapeDtypeStruct((KVH, S, HD), jnp.float32),
            jax.ShapeDtypeStruct((KVH, S, HD), jnp.float32),
        ],
    )(x2, s1, wq, wk, wv, qsf, ksf, gq, gq.T, gk, gk.T, cos2, sin2)


QBLK = 512
KBLK = 512
NQB = S // QBLK


def _attn_body(q_ref, k_ref, v_ref, o_ref):
    i = pl.program_id(1)
    q = (q_ref[0].astype(jnp.float32) * (1.0 / math.sqrt(HD))).astype(
        jnp.bfloat16)

    def full_step(kb, carry):
        m, l, acc = carry
        kblk = k_ref[0, pl.ds(kb * KBLK, KBLK), :].astype(jnp.bfloat16)
        vblk = v_ref[0, pl.ds(kb * KBLK, KBLK), :].astype(jnp.bfloat16)
        s = lax.dot_general(q, kblk, (((1,), (1,)), ((), ())),
                            preferred_element_type=jnp.float32)
        mn = jnp.maximum(m, jnp.max(s, axis=-1, keepdims=True))
        p = jnp.exp(s - mn)
        corr = jnp.exp(m - mn)
        l = l * corr + jnp.sum(p, axis=-1, keepdims=True)
        acc = acc * corr + jnp.dot(p.astype(jnp.bfloat16), vblk,
                                   preferred_element_type=jnp.float32)
        return mn, l, acc

    m0 = jnp.full((QBLK, 1), -1e30, jnp.float32)
    l0 = jnp.zeros((QBLK, 1), jnp.float32)
    a0 = jnp.zeros((QBLK, HD), jnp.float32)
    m, l, acc = lax.fori_loop(0, i, full_step, (m0, l0, a0))

    kblk = k_ref[0, pl.ds(i * KBLK, KBLK), :].astype(jnp.bfloat16)
    vblk = v_ref[0, pl.ds(i * KBLK, KBLK), :].astype(jnp.bfloat16)
    s = lax.dot_general(q, kblk, (((1,), (1,)), ((), ())),
                        preferred_element_type=jnp.float32)
    tri = (lax.broadcasted_iota(jnp.int32, (QBLK, KBLK), 0)
           >= lax.broadcasted_iota(jnp.int32, (QBLK, KBLK), 1))
    s = jnp.where(tri, s, -1e30)
    mn = jnp.maximum(m, jnp.max(s, axis=-1, keepdims=True))
    p = jnp.exp(s - mn)
    corr = jnp.exp(m - mn)
    l = l * corr + jnp.sum(p, axis=-1, keepdims=True)
    acc = acc * corr + jnp.dot(p.astype(jnp.bfloat16), vblk,
                               preferred_element_type=jnp.float32)
    o_ref[0] = acc / l


def _attention(q3, k3, v3):
    return pl.pallas_call(
        _attn_body,
        grid=(H, NQB),
        in_specs=[
            pl.BlockSpec((1, QBLK, HD), lambda h, i: (h, i, 0)),
            pl.BlockSpec((1, S, HD), lambda h, i: (h // GRP, 0, 0)),
            pl.BlockSpec((1, S, HD), lambda h, i: (h // GRP, 0, 0)),
        ],
        out_specs=pl.BlockSpec((1, QBLK, HD), lambda h, i: (h, i, 0)),
        out_shape=jax.ShapeDtypeStruct((H, S, HD), jnp.float32),
    )(q3, k3, v3)


def _post_body(o_ref, wo_ref, x_ref, s2_ref, rw_ref, rb_ref,
               x1_ref, h2_ref, rt_ref):
    o_flat = jnp.concatenate([o_ref[hh] for hh in range(H)], axis=-1)
    x1 = x_ref[...] + jnp.dot(o_flat, wo_ref[...],
                              preferred_element_type=jnp.float32)
    x1_ref[...] = x1
    h2 = _rmsn(x1, s2_ref[...])
    h2_ref[...] = jnp.concatenate(
        [h2, jnp.zeros((SBLK, DP - D), jnp.float32)], axis=-1)
    logits = (jnp.dot(h2, rw_ref[...], preferred_element_type=jnp.float32)
              + rb_ref[...])
    eio = lax.broadcasted_iota(jnp.int32, (SBLK, E), 1)
    m1 = jnp.max(logits, axis=-1, keepdims=True)
    i1 = jnp.min(jnp.where(logits == m1, eio, E), axis=-1, keepdims=True)
    l2 = jnp.where(eio == i1, -jnp.inf, logits)
    m2 = jnp.max(l2, axis=-1, keepdims=True)
    i2 = jnp.min(jnp.where(l2 == m2, eio, E), axis=-1, keepdims=True)
    g0 = 1.0 / (1.0 + jnp.exp(m2 - m1))
    g1 = 1.0 - g0
    z = jnp.zeros((SBLK, E - 4), jnp.float32)
    rt_ref[...] = jnp.concatenate(
        [i1.astype(jnp.float32), i2.astype(jnp.float32), g0, g1, z],
        axis=-1)


def _post_attn(o3, wo, x2, s2, rw, rb):
    return pl.pallas_call(
        _post_body,
        grid=(NSB,),
        in_specs=[
            pl.BlockSpec((H, SBLK, HD), lambda i: (0, i, 0)),
            pl.BlockSpec((H * HD, D), lambda i: (0, 0)),
            pl.BlockSpec((SBLK, D), lambda i: (i, 0)),
            pl.BlockSpec((1, D), lambda i: (0, 0)),
            pl.BlockSpec((D, E), lambda i: (0, 0)),
            pl.BlockSpec((1, E), lambda i: (0, 0)),
        ],
        out_specs=[
            pl.BlockSpec((SBLK, D), lambda i: (i, 0)),
            pl.BlockSpec((SBLK, DP), lambda i: (i, 0)),
            pl.BlockSpec((SBLK, E), lambda i: (i, 0)),
        ],
        out_shape=[
            jax.ShapeDtypeStruct((S, D), jnp.float32),
            jax.ShapeDtypeStruct((S, DP), jnp.float32),
            jax.ShapeDtypeStruct((S, E), jnp.float32),
        ],
    )(o3, wo, x2, s2, rw, rb)


def _excl_cumsum_rows(x):
    acc = x
    sh = 1
    while sh < S:
        acc = acc + jnp.concatenate(
            [jnp.zeros((sh, E), x.dtype), acc[:S - sh, :]], axis=0)
        sh *= 2
    return acc - x


def _excl_cumsum_lanes(x):
    acc = x
    for sh in (1, 2, 4):
        acc = acc + jnp.concatenate(
            [jnp.zeros((1, sh), x.dtype), acc[:, :E - sh]], axis=1)
    return acc - x


def _route_body(rt_ref, pos_ref, binfo_ref):
    rt = rt_ref[...]
    e0 = rt[:, 0:1].astype(jnp.int32)
    e1 = rt[:, 1:2].astype(jnp.int32)
    eio = lax.broadcasted_iota(jnp.int32, (S, E), 1)
    oh0 = (e0 == eio).astype(jnp.float32)
    oh1 = (e1 == eio).astype(jnp.float32)
    c0 = _excl_cumsum_rows(oh0)
    c1 = _excl_cumsum_rows(oh1)
    tot0 = jnp.sum(oh0, axis=0, keepdims=True)
    sizes = tot0 + jnp.sum(oh1, axis=0, keepdims=True)
    padded = jnp.floor((sizes + (BLK_M - 1)) / BLK_M) * BLK_M
    poff = _excl_cumsum_lanes(padded)
    pos0 = jnp.sum(oh0 * (poff + c0), axis=-1, keepdims=True)
    pos1 = jnp.sum(oh1 * (poff + tot0 + c1), axis=-1, keepdims=True)
    zs = jnp.zeros((S, E - 2), jnp.float32)
    pos_ref[...] = jnp.concatenate([pos0, pos1, zs], axis=-1).astype(jnp.int32)

    bstart = poff / BLK_M
    pb = padded / BLK_M
    nused = jnp.sum(pb)
    eiof = lax.broadcasted_iota(jnp.int32, (1, E), 1).astype(jnp.float32)
    belast = jnp.max(jnp.where(pb > 0, eiof, 0.0))
    bio = lax.broadcasted_iota(jnp.int32, (1, 128), 1).astype(jnp.float32)
    bi = jnp.zeros((1, 128), jnp.float32)
    for e in range(E):
        m = (bio >= bstart[0, e]) & (bio < bstart[0, e] + pb[0, e])
        bi = jnp.where(m, float(e), bi)
    bi = jnp.where((bio >= nused) & (bio < float(NB)), belast, bi)
    bi = jnp.where(bio == 32.0, nused, bi)
    binfo_ref[...] = bi.astype(jnp.int32)


def _route_plan(rt):
    return pl.pallas_call(
        _route_body,
        out_shape=[
            jax.ShapeDtypeStruct((S, E), jnp.int32),
            jax.ShapeDtypeStruct((1, 128), jnp.int32),
        ],
    )(rt)


_APW = TOPK * S // NW


@functools.lru_cache(maxsize=None)
def _sc_mesh():
    return plsc.VectorSubcoreMesh(core_axis_name="c", subcore_axis_name="s")


@functools.lru_cache(maxsize=None)
def _build_sc_dispatch():
    @functools.partial(
        pl.kernel,
        mesh=_sc_mesh(),
        out_type=jax.ShapeDtypeStruct((NBUF, DP), jnp.float32),
        scratch_types=[
            pltpu.VMEM((_APW,), jnp.int32),
            pltpu.VMEM((_APW, DP), jnp.float32),
            pltpu.SemaphoreType.DMA,
        ],
    )
    def dispatch(pos_hbm, h2_hbm, xs_hbm, idx_v, rows_v, sem):
        wid = lax.axis_index("s") * 2 + lax.axis_index("c")
        base = wid * _APW
        tbase = lax.rem(base, S)
        pltpu.sync_copy(pos_hbm.at[pl.ds(base, _APW)], idx_v)
        pltpu.sync_copy(h2_hbm.at[pl.ds(tbase, _APW)], rows_v)
        pltpu.async_copy(rows_v, xs_hbm.at[idx_v], sem).wait()

    return dispatch


def _sc_dispatch(pos_flat, h2):
    return _build_sc_dispatch()(pos_flat, h2)


_TPW = S // NW
_CH = 32


@functools.lru_cache(maxsize=None)
def _build_sc_combine():
    @functools.partial(
        pl.kernel,
        mesh=_sc_mesh(),
        out_type=(
            jax.ShapeDtypeStruct((S, DP), jnp.float32),
            jax.ShapeDtypeStruct((S, DP), jnp.float32),
        ),
        scratch_types=[
            pltpu.VMEM((_CH,), jnp.int32),
            pltpu.VMEM((_CH,), jnp.int32),
            pltpu.VMEM((_CH, DP), jnp.float32),
            pltpu.VMEM((_CH, DP), jnp.float32),
            pltpu.SemaphoreType.DMA,
            pltpu.SemaphoreType.DMA,
        ],
    )
    def combine(p0_hbm, p1_hbm, eo_hbm, r0_hbm, r1_hbm,
                i0_v, i1_v, r0_v, r1_v, s0, s1):
        wid = lax.axis_index("s") * 2 + lax.axis_index("c")
        for c in range(_TPW // _CH):
            base = wid * _TPW + c * _CH
            pltpu.sync_copy(p0_hbm.at[pl.ds(base, _CH)], i0_v)
            pltpu.sync_copy(p1_hbm.at[pl.ds(base, _CH)], i1_v)
            a = pltpu.async_copy(eo_hbm.at[i0_v], r0_v, s0)
            b = pltpu.async_copy(eo_hbm.at[i1_v], r1_v, s1)
            a.wait()
            b.wait()
            pltpu.sync_copy(r0_v, r0_hbm.at[pl.ds(base, _CH)])
            pltpu.sync_copy(r1_v, r1_hbm.at[pl.ds(base, _CH)])

    return combine


def _sc_combine(pos0, pos1, eo):
    return _build_sc_combine()(pos0, pos1, eo)


def _ffn_body(b_ref, xs_ref, w1_ref, b1_ref, w2_ref, b2_ref, eo_ref):
    i = pl.program_id(0)
    nused = b_ref[32]

    @pl.when(i < nused)
    def _():
        hid = (jnp.dot(xs_ref[...][:, :D], w1_ref[...],
                       preferred_element_type=jnp.float32) + b1_ref[0])
        act = jnp.square(jnp.maximum(hid, 0.0))
        contrib = (jnp.dot(act, w2_ref[...], preferred_element_type=jnp.float32)
                   + b2_ref[0])
        zpad = jnp.zeros((BLK_M, DP - D), jnp.float32)
        eo_ref[...] = jnp.concatenate([contrib, zpad], axis=-1)


def _expert_ffn(binfo, xs, w1, b1, w2, b2):
    grid_spec = pltpu.PrefetchScalarGridSpec(
        num_scalar_prefetch=1,
        grid=(NB,),
        in_specs=[
            pl.BlockSpec((BLK_M, DP), lambda i, b: (i, 0)),
            pl.BlockSpec((D, DFF), lambda i, b: (b[i], 0)),
            pl.BlockSpec((1, 1, DFF), lambda i, b: (b[i], 0, 0)),
            pl.BlockSpec((DFF, D), lambda i, b: (b[i], 0)),
            pl.BlockSpec((1, 1, D), lambda i, b: (b[i], 0, 0)),
        ],
        out_specs=pl.BlockSpec((BLK_M, DP), lambda i, b: (i, 0)),
    )
    return pl.pallas_call(
        _ffn_body,
        grid_spec=grid_spec,
        out_shape=jax.ShapeDtypeStruct((NBUF, DP), jnp.float32),
    )(binfo, xs, w1.reshape(E * D, DFF), b1.reshape(E, 1, DFF),
      w2.reshape(E * DFF, D), b2.reshape(E, 1, D))


def _comb_body(x1_ref, r0_ref, r1_ref, rt_ref, y_ref):
    g0 = rt_ref[:, 2:3]
    g1 = rt_ref[:, 3:4]
    y_ref[...] = (x1_ref[...] + g0 * r0_ref[...][:, :D]
                  + g1 * r1_ref[...][:, :D])


def _final_combine(x1, r0, r1, rt):
    return pl.pallas_call(
        _comb_body,
        grid=(NSB,),
        in_specs=[
            pl.BlockSpec((SBLK, D), lambda i: (i, 0)),
            pl.BlockSpec((SBLK, DP), lambda i: (i, 0)),
            pl.BlockSpec((SBLK, DP), lambda i: (i, 0)),
            pl.BlockSpec((SBLK, E), lambda i: (i, 0)),
        ],
        out_specs=pl.BlockSpec((SBLK, D), lambda i: (i, 0)),
        out_shape=jax.ShapeDtypeStruct((S, D), jnp.float32),
    )(x1, r0, r1, rt)


def kernel(x, rms1_scale, wq, wk, wv, wo, q_norm_scale, k_norm_scale,
           rms2_scale, router_w, router_b, w1, b1, w2, b2):
    x2 = x.reshape(S, D)
    s1 = rms1_scale.reshape(1, D)
    s2 = rms2_scale.reshape(1, D)
    rb = router_b.reshape(1, E)
    qsf = jnp.tile(q_norm_scale, H).reshape(1, H * HD)
    ksf = jnp.tile(k_norm_scale, KVH).reshape(1, KVH * HD)
    gq = jnp.repeat(jnp.eye(H, dtype=jnp.float32), HD, axis=0)
    gk = jnp.repeat(jnp.eye(KVH, dtype=jnp.float32), HD, axis=0)

    inv = 1.0 / (10000.0 ** (jnp.arange(HALF, dtype=jnp.float32) / HALF))
    ang = jnp.arange(S, dtype=jnp.float32)[:, None] * inv[None, :]
    cos = jnp.cos(ang)
    sin = jnp.sin(ang)
    cos2 = jnp.concatenate([cos, cos], axis=1)
    sin2 = jnp.concatenate([-sin, sin], axis=1)

    q3, k3, v3 = _qkv_proj(x2, s1, wq, wk, wv, qsf, ksf, gq, gk, cos2, sin2)
    o3 = _attention(q3, k3, v3)
    x1, h2, rt = _post_attn(o3, wo, x2, s2, router_w, rb)

    pos, binfo = _route_plan(rt)
    pos_flat = jnp.concatenate([pos[:, 0], pos[:, 1]], axis=0)
    xs = _sc_dispatch(pos_flat, h2)
    eo = _expert_ffn(binfo.reshape(128), xs, w1, b1, w2, b2)
    r0, r1 = _sc_combine(pos[:, 0], pos[:, 1], eo)
    y = _final_combine(x1, r0, r1, rt)
    return y.reshape(1, S, D)
